# Initial kernel scaffold; baseline (speedup 1.0000x reference)
import jax
import jax.numpy as jnp
from jax.experimental import pallas as pl


def kernel(x, edge_index, edge_w, params):
    raise NotImplementedError("write your pallas kernel here")


# speedup vs baseline: 1.7226x; 1.7226x over previous
import functools
import jax
import jax.numpy as jnp
from jax import lax
from jax.experimental import pallas as pl
from jax.experimental.pallas import tpu as pltpu
from jax.experimental.pallas import tpu_sc as plsc

N = 10000
E = 320000
GD = 128
EID = 16
F32 = jnp.float32

NC, NS = 2, 16
NW = NC * NS
EPW = E // NW
CH = 80
NCH = EPW // CH
NP = 10240
RPT = NP // NS

BE = 512
GE = E // BE
BN = 2000
GN = N // BN

@functools.lru_cache(maxsize=None)
def _sc_kernels():
    mesh = plsc.VectorSubcoreMesh(core_axis_name="c", subcore_axis_name="s",
                                  num_cores=NC, num_subcores=NS)

    @functools.partial(
        pl.kernel,
        out_type=[jax.ShapeDtypeStruct((E, 2 * GD), F32),
                  jax.ShapeDtypeStruct((E, GD), F32)],
        mesh=mesh,
        scratch_types=[
            pltpu.VMEM((CH,), jnp.int32),
            pltpu.VMEM((CH,), jnp.int32),
            pltpu.VMEM((CH, 2 * GD), F32),
            pltpu.VMEM((CH, GD), F32),
            pltpu.SemaphoreType.DMA,
        ],
    )
    def sc_gather(trow, tcol, row, col, g1o, g2o, idx_r, idx_c, g1v, g2v, sem):
        c = lax.axis_index("c")
        s = lax.axis_index("s")
        base = (s * NC + c) * EPW

        def body(i, carry):
            off = base + i * CH
            pltpu.sync_copy(row.at[pl.ds(off, CH)], idx_r)
            pltpu.sync_copy(col.at[pl.ds(off, CH)], idx_c)
            cp1 = pltpu.async_copy(trow.at[idx_r], g1v, sem)
            cp2 = pltpu.async_copy(tcol.at[idx_c], g2v, sem)
            cp1.wait()
            cp2.wait()
            pltpu.sync_copy(g1v, g1o.at[pl.ds(off, CH)])
            pltpu.sync_copy(g2v, g2o.at[pl.ds(off, CH)])
            return carry

        lax.fori_loop(0, NCH, body, 0)

    @functools.partial(
        pl.kernel,
        out_type=jax.ShapeDtypeStruct((NC * NP, GD), F32),
        mesh=mesh,
        scratch_types=[
            pltpu.VMEM_SHARED((NP, GD), F32),
            pltpu.VMEM((CH,), jnp.int32),
            pltpu.VMEM((CH, GD), F32),
        ],
    )
    def sc_count(colh, zz, onesh, outs, shared, idx, ones_v):
        c = lax.axis_index("c")
        s = lax.axis_index("s")
        base = (s * NC + c) * EPW
        pltpu.sync_copy(zz.at[pl.ds(s * RPT, RPT)],
                        shared.at[pl.ds(s * RPT, RPT)])
        pltpu.sync_copy(onesh, ones_v)
        plsc.subcore_barrier()

        def body(i, carry):
            off = base + i * CH
            pltpu.sync_copy(colh.at[pl.ds(off, CH)], idx)
            pltpu.sync_copy(ones_v, shared.at[idx], add=True)
            return carry

        lax.fori_loop(0, NCH, body, 0)
        plsc.subcore_barrier()
        pltpu.sync_copy(shared.at[pl.ds(s * RPT, RPT)],
                        outs.at[pl.ds(c * NP + s * RPT, RPT)])

    @functools.partial(
        pl.kernel,
        out_type=jax.ShapeDtypeStruct((NC * NP, GD), F32),
        mesh=mesh,
        scratch_types=[
            pltpu.VMEM_SHARED((NP, GD), F32),
            pltpu.VMEM((CH,), jnp.int32),
            pltpu.VMEM((CH, GD), F32),
        ],
    )
    def sc_scatter(n1, colh, zz, outs, shared, idx, vals):
        c = lax.axis_index("c")
        s = lax.axis_index("s")
        base = (s * NC + c) * EPW
        pltpu.sync_copy(zz.at[pl.ds(s * RPT, RPT)],
                        shared.at[pl.ds(s * RPT, RPT)])
        plsc.subcore_barrier()

        def body(i, carry):
            off = base + i * CH
            pltpu.sync_copy(colh.at[pl.ds(off, CH)], idx)
            pltpu.sync_copy(n1.at[pl.ds(off, CH)], vals)
            pltpu.sync_copy(vals, shared.at[idx], add=True)
            return carry

        lax.fori_loop(0, NCH, body, 0)
        plsc.subcore_barrier()
        pltpu.sync_copy(shared.at[pl.ds(s * RPT, RPT)],
                        outs.at[pl.ds(c * NP + s * RPT, RPT)])

    return sc_gather, sc_count, sc_scatter


def _dot(a, b):
    return jnp.dot(a, b, preferred_element_type=F32)


def _wspec(r, c=GD):
    return pl.BlockSpec((r, c), lambda i: (0, 0))


def _proj_body(x_ref, wr, wb, wc, trow_o, tcol_o):
    x = x_ref[...]
    trow_o[...] = jnp.concatenate([_dot(x, wr[...]), _dot(x, wb[...])], axis=1)
    tcol_o[...] = _dot(x, wc[...])


_tc_proj = pl.pallas_call(
    _proj_body,
    grid=(GN,),
    in_specs=[pl.BlockSpec((BN, GD), lambda i: (i, 0)),
              _wspec(GD), _wspec(GD), _wspec(GD)],
    out_specs=[pl.BlockSpec((BN, 2 * GD), lambda i: (i, 0)),
               pl.BlockSpec((BN, GD), lambda i: (i, 0))],
    out_shape=[jax.ShapeDtypeStruct((N, 2 * GD), F32),
               jax.ShapeDtypeStruct((N, GD), F32)],
)


def _make_edge(ewd, write_e):
    def body(g1, g2, ew, wew, w2e, wn1e, wn2, be1, b2e, bn1, bn2, *outs):
        g1v = g1[...]
        h1 = jnp.maximum(
            g1v[:, :GD] + g2[...] + _dot(ew[...], wew[...]) + be1[...], 0.0)
        e = _dot(h1, w2e[...]) + b2e[...]
        h2 = jnp.maximum(g1v[:, GD:] + _dot(e, wn1e[...]) + bn1[...], 0.0)
        n1v = _dot(h2, wn2[...]) + bn2[...]
        if write_e:
            outs[0][...] = e
            outs[1][...] = n1v
        else:
            outs[0][...] = n1v

    out_specs = [pl.BlockSpec((BE, GD), lambda i: (i, 0))]
    out_shape = [jax.ShapeDtypeStruct((E, GD), F32)]
    if write_e:
        out_specs = out_specs * 2
        out_shape = out_shape * 2
    return pl.pallas_call(
        body,
        grid=(GE,),
        in_specs=[pl.BlockSpec((BE, 2 * GD), lambda i: (i, 0)),
                  pl.BlockSpec((BE, GD), lambda i: (i, 0)),
                  pl.BlockSpec((BE, ewd), lambda i: (i, 0)),
                  _wspec(ewd), _wspec(GD), _wspec(GD), _wspec(GD),
                  _wspec(1), _wspec(1), _wspec(1), _wspec(1)],
        out_specs=out_specs,
        out_shape=out_shape,
    )


_tc_edge1 = _make_edge(EID, True)
_tc_edge2 = _make_edge(GD, True)
_tc_edge3 = _make_edge(GD, False)


def _make_node(first, last):
    def body(*refs):
        it = iter(refs)
        p0, p1 = next(it), next(it)
        if first:
            c0, c1 = next(it), next(it)
        else:
            inv_in = next(it)
        x_ref, wmx, wma, wm2, ubm, bm2 = (next(it) for _ in range(6))
        if not last:
            wr2, wb2, wc2 = next(it), next(it), next(it)
            xn_o, trow_o, tcol_o = next(it), next(it), next(it)
        mp_o = next(it)
        if first:
            inv_o = next(it)

        i = pl.program_id(0)
        s = p0[0] + p1[0]
        if first:
            cnt = c0[0][:, :1] + c1[0][:, :1]
            invb = 1.0 / jnp.maximum(cnt, 1.0)
            inv_o[...] = jnp.broadcast_to(invb, (BN, GD))
            agg = s * invb
        else:
            agg = s * inv_in[...]
        x = x_ref[...]
        h = jnp.maximum(_dot(x, wmx[...]) + _dot(agg, wma[...]) + ubm[...], 0.0)
        xn = _dot(h, wm2[...]) + bm2[...]
        if not last:
            xn_o[...] = xn
            trow_o[...] = jnp.concatenate(
                [_dot(xn, wr2[...]), _dot(xn, wb2[...])], axis=1)
            tcol_o[...] = _dot(xn, wc2[...])

        @pl.when(i == 0)
        def _():
            mp_o[...] = jnp.zeros((8, GD), F32)

        mp_o[0:1, :] = mp_o[0:1, :] + jnp.sum(xn, axis=0, keepdims=True)

    p_spec = [pl.BlockSpec((1, BN, GD), lambda i: (0, i, 0)),
              pl.BlockSpec((1, BN, GD), lambda i: (1, i, 0))]
    in_specs = list(p_spec)
    if first:
        in_specs += [pl.BlockSpec((1, BN, GD), lambda i: (0, i, 0)),
                     pl.BlockSpec((1, BN, GD), lambda i: (1, i, 0))]
    else:
        in_specs += [pl.BlockSpec((BN, GD), lambda i: (i, 0))]
    in_specs += [pl.BlockSpec((BN, GD), lambda i: (i, 0)),
                 _wspec(GD), _wspec(GD), _wspec(GD), _wspec(1), _wspec(1)]
    if not last:
        in_specs += [_wspec(GD), _wspec(GD), _wspec(GD)]

    out_specs, out_shape = [], []
    if not last:
        out_specs += [pl.BlockSpec((BN, GD), lambda i: (i, 0)),
                      pl.BlockSpec((BN, 2 * GD), lambda i: (i, 0)),
                      pl.BlockSpec((BN, GD), lambda i: (i, 0))]
        out_shape += [jax.ShapeDtypeStruct((N, GD), F32),
                      jax.ShapeDtypeStruct((N, 2 * GD), F32),
                      jax.ShapeDtypeStruct((N, GD), F32)]
    out_specs += [pl.BlockSpec((8, GD), lambda i: (0, 0))]
    out_shape += [jax.ShapeDtypeStruct((8, GD), F32)]
    if first:
        out_specs += [pl.BlockSpec((BN, GD), lambda i: (i, 0))]
        out_shape += [jax.ShapeDtypeStruct((N, GD), F32)]
    return pl.pallas_call(body, grid=(GN,), in_specs=in_specs,
                          out_specs=out_specs, out_shape=out_shape)


_tc_node1 = _make_node(True, False)
_tc_node2 = _make_node(False, False)
_tc_node3 = _make_node(False, True)


def _make_glob(last):
    def body(mp, u, wgu, wgm, bg1, wg2, bg2, *rest):
        m = jnp.sum(mp[...], axis=0, keepdims=True) * (1.0 / N)
        m8 = jnp.broadcast_to(m, (8, GD))
        h = jnp.maximum(_dot(u[...], wgu[...]) + _dot(m8, wgm[...]) + bg1[...],
                        0.0)
        un = _dot(h, wg2[...]) + bg2[...]
        if last:
            wlin, blin, out_o = rest
            out_o[...] = (_dot(un, wlin[...]) + blin[...])[0:1, :]
        else:
            wue, b1e, wum, bm1, un_o, be1_o, ubm_o = rest
            un_o[...] = un
            be1_o[...] = (_dot(un, wue[...]) + b1e[...])[0:1, :]
            ubm_o[...] = (_dot(un, wum[...]) + bm1[...])[0:1, :]

    in_specs = [_wspec(8), _wspec(8), _wspec(GD), _wspec(GD), _wspec(1),
                _wspec(GD), _wspec(1)]
    if last:
        in_specs += [_wspec(GD), _wspec(1)]
        out_specs = [_wspec(1)]
        out_shape = [jax.ShapeDtypeStruct((1, GD), F32)]
    else:
        in_specs += [_wspec(GD), _wspec(1), _wspec(GD), _wspec(1)]
        out_specs = [_wspec(8), _wspec(1), _wspec(1)]
        out_shape = [jax.ShapeDtypeStruct((8, GD), F32),
                     jax.ShapeDtypeStruct((1, GD), F32),
                     jax.ShapeDtypeStruct((1, GD), F32)]
    return pl.pallas_call(body, grid=(1,), in_specs=in_specs,
                          out_specs=out_specs, out_shape=out_shape)


_tc_glob_mid = _make_glob(False)
_tc_glob_last = _make_glob(True)


def _split(p, ni, ewd, has_u):
    (w1e, b1e, w2e, b2e) = p['edge']
    (wn1, bn1, wn2, bn2) = p['nm1']
    (wm1, bm1, wm2, bm2) = p['nm2']
    (wg1, bg1, wg2, bg2) = p['glob']
    r = lambda b: b.reshape(1, GD)
    d = {
        'wr': w1e[:, :ni].T, 'wc': w1e[:, ni:2 * ni].T,
        'wew': w1e[:, 2 * ni:2 * ni + ewd].T,
        'b1e': r(b1e), 'w2e': w2e.T, 'b2e': r(b2e),
        'wbx': wn1[:, :ni].T, 'wn1e': wn1[:, ni:].T, 'bn1': r(bn1),
        'wn2': wn2.T, 'bn2': r(bn2),
        'wmx': wm1[:, :ni].T, 'wma': wm1[:, ni:ni + GD].T, 'bm1': r(bm1),
        'wm2': wm2.T, 'bm2': r(bm2),
        'wgu': wg1[:, :GD].T, 'wgm': wg1[:, GD:].T, 'bg1': r(bg1),
        'wg2': wg2.T, 'bg2': r(bg2),
    }
    if has_u:
        d['wue'] = w1e[:, 2 * ni + ewd:].T
        d['wum'] = wm1[:, ni + GD:].T
    return d


def kernel(x, edge_index, edge_w, params):
    _sc_gather, _sc_count, _sc_scatter = _sc_kernels()
    row, col = edge_index[0], edge_index[1]
    q1 = _split(params['conv1'], GD, EID, False)
    q2 = _split(params['conv2'], GD, GD, True)
    q3 = _split(params['conv3'], GD, GD, True)
    wlin, blin = params['lin'][0].T, params['lin'][1].reshape(1, GD)

    u0 = jax.random.normal(jax.random.key(42), (1, GD), F32) * 0.01
    u0p = jnp.concatenate([u0, jnp.zeros((7, GD), F32)], axis=0)
    zz = jnp.zeros((NP, GD), F32)
    onesh = jnp.ones((CH, GD), F32)

    trow, tcol = _tc_proj(x, q1['wr'], q1['wbx'], q1['wc'])
    g1, g2 = _sc_gather(trow, tcol, row, col)
    e1, n1 = _tc_edge1(g1, g2, edge_w, q1['wew'], q1['w2e'], q1['wn1e'],
                       q1['wn2'], q1['b1e'], q1['b2e'], q1['bn1'], q1['bn2'])
    pc = _sc_count(col, zz, onesh)
    ps = _sc_scatter(n1, col, zz)
    xn1, trow, tcol, mp, inv = _tc_node1(
        ps.reshape(2, NP, GD), ps.reshape(2, NP, GD),
        pc.reshape(2, NP, GD), pc.reshape(2, NP, GD),
        x, q1['wmx'], q1['wma'], q1['wm2'], q1['bm1'], q1['bm2'],
        q2['wr'], q2['wbx'], q2['wc'])
    u1, be1_2, ubm2 = _tc_glob_mid(mp, u0p, q1['wgu'], q1['wgm'], q1['bg1'],
                                   q1['wg2'], q1['bg2'],
                                   q2['wue'], q2['b1e'], q2['wum'], q2['bm1'])

    g1, g2 = _sc_gather(trow, tcol, row, col)
    e2, n2 = _tc_edge2(g1, g2, e1, q2['wew'], q2['w2e'], q2['wn1e'],
                       q2['wn2'], be1_2, q2['b2e'], q2['bn1'], q2['bn2'])
    ps = _sc_scatter(n2, col, zz)
    xn2, trow, tcol, mp = _tc_node2(
        ps.reshape(2, NP, GD), ps.reshape(2, NP, GD), inv,
        xn1, q2['wmx'], q2['wma'], q2['wm2'], ubm2, q2['bm2'],
        q3['wr'], q3['wbx'], q3['wc'])
    u2, be1_3, ubm3 = _tc_glob_mid(mp, u1, q2['wgu'], q2['wgm'], q2['bg1'],
                                   q2['wg2'], q2['bg2'],
                                   q3['wue'], q3['b1e'], q3['wum'], q3['bm1'])

    g1, g2 = _sc_gather(trow, tcol, row, col)
    (n3,) = _tc_edge3(g1, g2, e2, q3['wew'], q3['w2e'], q3['wn1e'],
                      q3['wn2'], be1_3, q3['b2e'], q3['bn1'], q3['bn2'])
    ps = _sc_scatter(n3, col, zz)
    (mp,) = _tc_node3(ps.reshape(2, NP, GD), ps.reshape(2, NP, GD), inv,
                      xn2, q3['wmx'], q3['wma'], q3['wm2'], ubm3, q3['bm2'])
    (out,) = _tc_glob_last(mp, u2, q3['wgu'], q3['wgm'], q3['bg1'],
                           q3['wg2'], q3['bg2'], wlin, blin)
    return out


# speedup vs baseline: 1.7524x; 1.0173x over previous
import functools
import jax
import jax.numpy as jnp
from jax import lax
from jax.experimental import pallas as pl
from jax.experimental.pallas import tpu as pltpu
from jax.experimental.pallas import tpu_sc as plsc

N = 10000
E = 320000
GD = 128
EID = 16
F32 = jnp.float32

NC, NS = 2, 16
NW = NC * NS
EPW = E // NW
CH = 40
NCH = EPW // CH
NG = NCH // 2
NP = 10240
RPT = NP // NS

BE = 512
GE = E // BE
BN = 2000
GN = N // BN

@functools.lru_cache(maxsize=None)
def _sc_kernels():
    mesh = plsc.VectorSubcoreMesh(core_axis_name="c", subcore_axis_name="s",
                                  num_cores=NC, num_subcores=NS)

    @functools.partial(
        pl.kernel,
        out_type=[jax.ShapeDtypeStruct((E, 2 * GD), F32),
                  jax.ShapeDtypeStruct((E, GD), F32)],
        mesh=mesh,
        scratch_types=[
            pltpu.VMEM((NCH, CH), jnp.int32),
            pltpu.VMEM((NCH, CH), jnp.int32),
            pltpu.VMEM((CH, 2 * GD), F32),
            pltpu.VMEM((CH, 2 * GD), F32),
            pltpu.VMEM((CH, GD), F32),
            pltpu.VMEM((CH, GD), F32),
            pltpu.SemaphoreType.DMA,
            pltpu.SemaphoreType.DMA,
            pltpu.SemaphoreType.DMA,
            pltpu.SemaphoreType.DMA,
        ],
    )
    def sc_gather(trow, tcol, rowr, colr, g1o, g2o,
                  idx_r, idx_c, g1a, g1b, g2a, g2b, sga, sgb, swa, swb):
        c = lax.axis_index("c")
        s = lax.axis_index("s")
        w = s * NC + c
        base = w * EPW
        pltpu.sync_copy(rowr.at[w], idx_r)
        pltpu.sync_copy(colr.at[w], idx_c)

        def body(g, carry):
            i0 = 2 * g
            i1 = 2 * g + 1
            off0 = base + i0 * CH
            off1 = base + i1 * CH

            @pl.when(g >= 1)
            def _():
                pltpu.make_async_copy(g1a, g1o.at[pl.ds(off0, CH)], swa).wait()
                pltpu.make_async_copy(g2a, g2o.at[pl.ds(off0, CH)], swa).wait()
            cpa1 = pltpu.async_copy(trow.at[idx_r.at[i0]], g1a, sga)
            cpa2 = pltpu.async_copy(tcol.at[idx_c.at[i0]], g2a, sga)

            @pl.when(g >= 1)
            def _():
                pltpu.make_async_copy(g1b, g1o.at[pl.ds(off1, CH)], swb).wait()
                pltpu.make_async_copy(g2b, g2o.at[pl.ds(off1, CH)], swb).wait()
            cpb1 = pltpu.async_copy(trow.at[idx_r.at[i1]], g1b, sgb)
            cpb2 = pltpu.async_copy(tcol.at[idx_c.at[i1]], g2b, sgb)

            cpa1.wait()
            cpa2.wait()
            pltpu.async_copy(g1a, g1o.at[pl.ds(off0, CH)], swa)
            pltpu.async_copy(g2a, g2o.at[pl.ds(off0, CH)], swa)
            cpb1.wait()
            cpb2.wait()
            pltpu.async_copy(g1b, g1o.at[pl.ds(off1, CH)], swb)
            pltpu.async_copy(g2b, g2o.at[pl.ds(off1, CH)], swb)
            return carry

        lax.fori_loop(0, NG, body, 0)
        pltpu.make_async_copy(g1a, g1o.at[pl.ds(base, CH)], swa).wait()
        pltpu.make_async_copy(g2a, g2o.at[pl.ds(base, CH)], swa).wait()
        pltpu.make_async_copy(g1b, g1o.at[pl.ds(base, CH)], swb).wait()
        pltpu.make_async_copy(g2b, g2o.at[pl.ds(base, CH)], swb).wait()

    @functools.partial(
        pl.kernel,
        out_type=jax.ShapeDtypeStruct((NC * NP, GD), F32),
        mesh=mesh,
        scratch_types=[
            pltpu.VMEM_SHARED((NP, GD), F32),
            pltpu.VMEM((CH,), jnp.int32),
            pltpu.VMEM((CH, GD), F32),
        ],
    )
    def sc_count(colh, zz, onesh, outs, shared, idx, ones_v):
        c = lax.axis_index("c")
        s = lax.axis_index("s")
        base = (s * NC + c) * EPW
        pltpu.sync_copy(zz.at[pl.ds(s * RPT, RPT)],
                        shared.at[pl.ds(s * RPT, RPT)])
        pltpu.sync_copy(onesh, ones_v)
        plsc.subcore_barrier()

        def body(i, carry):
            off = base + i * CH
            pltpu.sync_copy(colh.at[pl.ds(off, CH)], idx)
            pltpu.sync_copy(ones_v, shared.at[idx], add=True)
            return carry

        lax.fori_loop(0, NCH, body, 0)
        plsc.subcore_barrier()
        pltpu.sync_copy(shared.at[pl.ds(s * RPT, RPT)],
                        outs.at[pl.ds(c * NP + s * RPT, RPT)])

    @functools.partial(
        pl.kernel,
        out_type=jax.ShapeDtypeStruct((NC * NP, GD), F32),
        mesh=mesh,
        scratch_types=[
            pltpu.VMEM_SHARED((NP, GD), F32),
            pltpu.VMEM((CH,), jnp.int32),
            pltpu.VMEM((CH, GD), F32),
        ],
    )
    def sc_scatter(n1, colh, zz, outs, shared, idx, vals):
        c = lax.axis_index("c")
        s = lax.axis_index("s")
        base = (s * NC + c) * EPW
        pltpu.sync_copy(zz.at[pl.ds(s * RPT, RPT)],
                        shared.at[pl.ds(s * RPT, RPT)])
        plsc.subcore_barrier()

        def body(i, carry):
            off = base + i * CH
            pltpu.sync_copy(colh.at[pl.ds(off, CH)], idx)
            pltpu.sync_copy(n1.at[pl.ds(off, CH)], vals)
            pltpu.sync_copy(vals, shared.at[idx], add=True)
            return carry

        lax.fori_loop(0, NCH, body, 0)
        plsc.subcore_barrier()
        pltpu.sync_copy(shared.at[pl.ds(s * RPT, RPT)],
                        outs.at[pl.ds(c * NP + s * RPT, RPT)])

    return sc_gather, sc_count, sc_scatter


def _dot(a, b):
    return jnp.dot(a, b, preferred_element_type=F32)


def _wspec(r, c=GD):
    return pl.BlockSpec((r, c), lambda i: (0, 0))


def _proj_body(x_ref, wr, wb, wc, trow_o, tcol_o):
    x = x_ref[...]
    trow_o[...] = jnp.concatenate([_dot(x, wr[...]), _dot(x, wb[...])], axis=1)
    tcol_o[...] = _dot(x, wc[...])


_tc_proj = pl.pallas_call(
    _proj_body,
    grid=(GN,),
    in_specs=[pl.BlockSpec((BN, GD), lambda i: (i, 0)),
              _wspec(GD), _wspec(GD), _wspec(GD)],
    out_specs=[pl.BlockSpec((BN, 2 * GD), lambda i: (i, 0)),
               pl.BlockSpec((BN, GD), lambda i: (i, 0))],
    out_shape=[jax.ShapeDtypeStruct((N, 2 * GD), F32),
               jax.ShapeDtypeStruct((N, GD), F32)],
)


def _make_edge(ewd, write_e):
    def body(g1, g2, ew, wew, w2e, wn1e, wn2, be1, b2e, bn1, bn2, *outs):
        g1v = g1[...]
        h1 = jnp.maximum(
            g1v[:, :GD] + g2[...] + _dot(ew[...], wew[...]) + be1[...], 0.0)
        e = _dot(h1, w2e[...]) + b2e[...]
        h2 = jnp.maximum(g1v[:, GD:] + _dot(e, wn1e[...]) + bn1[...], 0.0)
        n1v = _dot(h2, wn2[...]) + bn2[...]
        if write_e:
            outs[0][...] = e
            outs[1][...] = n1v
        else:
            outs[0][...] = n1v

    out_specs = [pl.BlockSpec((BE, GD), lambda i: (i, 0))]
    out_shape = [jax.ShapeDtypeStruct((E, GD), F32)]
    if write_e:
        out_specs = out_specs * 2
        out_shape = out_shape * 2
    return pl.pallas_call(
        body,
        grid=(GE,),
        in_specs=[pl.BlockSpec((BE, 2 * GD), lambda i: (i, 0)),
                  pl.BlockSpec((BE, GD), lambda i: (i, 0)),
                  pl.BlockSpec((BE, ewd), lambda i: (i, 0)),
                  _wspec(ewd), _wspec(GD), _wspec(GD), _wspec(GD),
                  _wspec(1), _wspec(1), _wspec(1), _wspec(1)],
        out_specs=out_specs,
        out_shape=out_shape,
    )


_tc_edge1 = _make_edge(EID, True)
_tc_edge2 = _make_edge(GD, True)
_tc_edge3 = _make_edge(GD, False)


def _make_node(first, last):
    def body(*refs):
        it = iter(refs)
        p0, p1 = next(it), next(it)
        if first:
            c0, c1 = next(it), next(it)
        else:
            inv_in = next(it)
        x_ref, wmx, wma, wm2, ubm, bm2 = (next(it) for _ in range(6))
        if not last:
            wr2, wb2, wc2 = next(it), next(it), next(it)
            xn_o, trow_o, tcol_o = next(it), next(it), next(it)
        mp_o = next(it)
        if first:
            inv_o = next(it)

        i = pl.program_id(0)
        s = p0[0] + p1[0]
        if first:
            cnt = c0[0][:, :1] + c1[0][:, :1]
            invb = 1.0 / jnp.maximum(cnt, 1.0)
            inv_o[...] = jnp.broadcast_to(invb, (BN, GD))
            agg = s * invb
        else:
            agg = s * inv_in[...]
        x = x_ref[...]
        h = jnp.maximum(_dot(x, wmx[...]) + _dot(agg, wma[...]) + ubm[...], 0.0)
        xn = _dot(h, wm2[...]) + bm2[...]
        if not last:
            xn_o[...] = xn
            trow_o[...] = jnp.concatenate(
                [_dot(xn, wr2[...]), _dot(xn, wb2[...])], axis=1)
            tcol_o[...] = _dot(xn, wc2[...])

        @pl.when(i == 0)
        def _():
            mp_o[...] = jnp.zeros((8, GD), F32)

        mp_o[0:1, :] = mp_o[0:1, :] + jnp.sum(xn, axis=0, keepdims=True)

    p_spec = [pl.BlockSpec((1, BN, GD), lambda i: (0, i, 0)),
              pl.BlockSpec((1, BN, GD), lambda i: (1, i, 0))]
    in_specs = list(p_spec)
    if first:
        in_specs += [pl.BlockSpec((1, BN, GD), lambda i: (0, i, 0)),
                     pl.BlockSpec((1, BN, GD), lambda i: (1, i, 0))]
    else:
        in_specs += [pl.BlockSpec((BN, GD), lambda i: (i, 0))]
    in_specs += [pl.BlockSpec((BN, GD), lambda i: (i, 0)),
                 _wspec(GD), _wspec(GD), _wspec(GD), _wspec(1), _wspec(1)]
    if not last:
        in_specs += [_wspec(GD), _wspec(GD), _wspec(GD)]

    out_specs, out_shape = [], []
    if not last:
        out_specs += [pl.BlockSpec((BN, GD), lambda i: (i, 0)),
                      pl.BlockSpec((BN, 2 * GD), lambda i: (i, 0)),
                      pl.BlockSpec((BN, GD), lambda i: (i, 0))]
        out_shape += [jax.ShapeDtypeStruct((N, GD), F32),
                      jax.ShapeDtypeStruct((N, 2 * GD), F32),
                      jax.ShapeDtypeStruct((N, GD), F32)]
    out_specs += [pl.BlockSpec((8, GD), lambda i: (0, 0))]
    out_shape += [jax.ShapeDtypeStruct((8, GD), F32)]
    if first:
        out_specs += [pl.BlockSpec((BN, GD), lambda i: (i, 0))]
        out_shape += [jax.ShapeDtypeStruct((N, GD), F32)]
    return pl.pallas_call(body, grid=(GN,), in_specs=in_specs,
                          out_specs=out_specs, out_shape=out_shape)


_tc_node1 = _make_node(True, False)
_tc_node2 = _make_node(False, False)
_tc_node3 = _make_node(False, True)


def _make_glob(last):
    def body(mp, u, wgu, wgm, bg1, wg2, bg2, *rest):
        m = jnp.sum(mp[...], axis=0, keepdims=True) * (1.0 / N)
        m8 = jnp.broadcast_to(m, (8, GD))
        h = jnp.maximum(_dot(u[...], wgu[...]) + _dot(m8, wgm[...]) + bg1[...],
                        0.0)
        un = _dot(h, wg2[...]) + bg2[...]
        if last:
            wlin, blin, out_o = rest
            out_o[...] = (_dot(un, wlin[...]) + blin[...])[0:1, :]
        else:
            wue, b1e, wum, bm1, un_o, be1_o, ubm_o = rest
            un_o[...] = un
            be1_o[...] = (_dot(un, wue[...]) + b1e[...])[0:1, :]
            ubm_o[...] = (_dot(un, wum[...]) + bm1[...])[0:1, :]

    in_specs = [_wspec(8), _wspec(8), _wspec(GD), _wspec(GD), _wspec(1),
                _wspec(GD), _wspec(1)]
    if last:
        in_specs += [_wspec(GD), _wspec(1)]
        out_specs = [_wspec(1)]
        out_shape = [jax.ShapeDtypeStruct((1, GD), F32)]
    else:
        in_specs += [_wspec(GD), _wspec(1), _wspec(GD), _wspec(1)]
        out_specs = [_wspec(8), _wspec(1), _wspec(1)]
        out_shape = [jax.ShapeDtypeStruct((8, GD), F32),
                     jax.ShapeDtypeStruct((1, GD), F32),
                     jax.ShapeDtypeStruct((1, GD), F32)]
    return pl.pallas_call(body, grid=(1,), in_specs=in_specs,
                          out_specs=out_specs, out_shape=out_shape)


_tc_glob_mid = _make_glob(False)
_tc_glob_last = _make_glob(True)


def _split(p, ni, ewd, has_u):
    (w1e, b1e, w2e, b2e) = p['edge']
    (wn1, bn1, wn2, bn2) = p['nm1']
    (wm1, bm1, wm2, bm2) = p['nm2']
    (wg1, bg1, wg2, bg2) = p['glob']
    r = lambda b: b.reshape(1, GD)
    d = {
        'wr': w1e[:, :ni].T, 'wc': w1e[:, ni:2 * ni].T,
        'wew': w1e[:, 2 * ni:2 * ni + ewd].T,
        'b1e': r(b1e), 'w2e': w2e.T, 'b2e': r(b2e),
        'wbx': wn1[:, :ni].T, 'wn1e': wn1[:, ni:].T, 'bn1': r(bn1),
        'wn2': wn2.T, 'bn2': r(bn2),
        'wmx': wm1[:, :ni].T, 'wma': wm1[:, ni:ni + GD].T, 'bm1': r(bm1),
        'wm2': wm2.T, 'bm2': r(bm2),
        'wgu': wg1[:, :GD].T, 'wgm': wg1[:, GD:].T, 'bg1': r(bg1),
        'wg2': wg2.T, 'bg2': r(bg2),
    }
    if has_u:
        d['wue'] = w1e[:, 2 * ni + ewd:].T
        d['wum'] = wm1[:, ni + GD:].T
    return d


def kernel(x, edge_index, edge_w, params):
    _sc_gather, _sc_count, _sc_scatter = _sc_kernels()
    row, col = edge_index[0], edge_index[1]
    q1 = _split(params['conv1'], GD, EID, False)
    q2 = _split(params['conv2'], GD, GD, True)
    q3 = _split(params['conv3'], GD, GD, True)
    wlin, blin = params['lin'][0].T, params['lin'][1].reshape(1, GD)

    u0 = jax.random.normal(jax.random.key(42), (1, GD), F32) * 0.01
    u0p = jnp.concatenate([u0, jnp.zeros((7, GD), F32)], axis=0)
    zz = jnp.zeros((NP, GD), F32)
    onesh = jnp.ones((CH, GD), F32)

    rowr = row.reshape(NW, NCH, CH)
    colr = col.reshape(NW, NCH, CH)
    trow, tcol = _tc_proj(x, q1['wr'], q1['wbx'], q1['wc'])
    g1, g2 = _sc_gather(trow, tcol, rowr, colr)
    e1, n1 = _tc_edge1(g1, g2, edge_w, q1['wew'], q1['w2e'], q1['wn1e'],
                       q1['wn2'], q1['b1e'], q1['b2e'], q1['bn1'], q1['bn2'])
    pc = _sc_count(col, zz, onesh)
    ps = _sc_scatter(n1, col, zz)
    xn1, trow, tcol, mp, inv = _tc_node1(
        ps.reshape(2, NP, GD), ps.reshape(2, NP, GD),
        pc.reshape(2, NP, GD), pc.reshape(2, NP, GD),
        x, q1['wmx'], q1['wma'], q1['wm2'], q1['bm1'], q1['bm2'],
        q2['wr'], q2['wbx'], q2['wc'])
    u1, be1_2, ubm2 = _tc_glob_mid(mp, u0p, q1['wgu'], q1['wgm'], q1['bg1'],
                                   q1['wg2'], q1['bg2'],
                                   q2['wue'], q2['b1e'], q2['wum'], q2['bm1'])

    g1, g2 = _sc_gather(trow, tcol, rowr, colr)
    e2, n2 = _tc_edge2(g1, g2, e1, q2['wew'], q2['w2e'], q2['wn1e'],
                       q2['wn2'], be1_2, q2['b2e'], q2['bn1'], q2['bn2'])
    ps = _sc_scatter(n2, col, zz)
    xn2, trow, tcol, mp = _tc_node2(
        ps.reshape(2, NP, GD), ps.reshape(2, NP, GD), inv,
        xn1, q2['wmx'], q2['wma'], q2['wm2'], ubm2, q2['bm2'],
        q3['wr'], q3['wbx'], q3['wc'])
    u2, be1_3, ubm3 = _tc_glob_mid(mp, u1, q2['wgu'], q2['wgm'], q2['bg1'],
                                   q2['wg2'], q2['bg2'],
                                   q3['wue'], q3['b1e'], q3['wum'], q3['bm1'])

    g1, g2 = _sc_gather(trow, tcol, rowr, colr)
    (n3,) = _tc_edge3(g1, g2, e2, q3['wew'], q3['w2e'], q3['wn1e'],
                      q3['wn2'], be1_3, q3['b2e'], q3['bn1'], q3['bn2'])
    ps = _sc_scatter(n3, col, zz)
    (mp,) = _tc_node3(ps.reshape(2, NP, GD), ps.reshape(2, NP, GD), inv,
                      xn2, q3['wmx'], q3['wma'], q3['wm2'], ubm3, q3['bm2'])
    (out,) = _tc_glob_last(mp, u2, q3['wgu'], q3['wgm'], q3['bg1'],
                           q3['wg2'], q3['bg2'], wlin, blin)
    return out


# speedup vs baseline: 2.0296x; 1.1582x over previous
import functools
import jax
import jax.numpy as jnp
from jax import lax
from jax.experimental import pallas as pl
from jax.experimental.pallas import tpu as pltpu
from jax.experimental.pallas import tpu_sc as plsc

N = 10000
E = 320000
GD = 128
EID = 16
F32 = jnp.float32

NC, NS = 2, 16
NW = NC * NS
EPW = E // NW
CH = 40
NCH = EPW // CH
NG = NCH // 2
NP = 10240
RPT = NP // NS

BE = 512
GE = E // BE
BN = 2000
GN = N // BN

@functools.lru_cache(maxsize=None)
def _sc_kernels():
    mesh = plsc.VectorSubcoreMesh(core_axis_name="c", subcore_axis_name="s",
                                  num_cores=NC, num_subcores=NS)

    @functools.partial(
        pl.kernel,
        out_type=[jax.ShapeDtypeStruct((E, 2 * GD), F32),
                  jax.ShapeDtypeStruct((E, GD), F32)],
        mesh=mesh,
        scratch_types=[
            pltpu.VMEM((NCH, CH), jnp.int32),
            pltpu.VMEM((NCH, CH), jnp.int32),
            pltpu.VMEM((CH, 2 * GD), F32),
            pltpu.VMEM((CH, 2 * GD), F32),
            pltpu.VMEM((CH, GD), F32),
            pltpu.VMEM((CH, GD), F32),
            pltpu.SemaphoreType.DMA,
            pltpu.SemaphoreType.DMA,
            pltpu.SemaphoreType.DMA,
            pltpu.SemaphoreType.DMA,
        ],
    )
    def sc_gather(trow, tcol, rowr, colr, g1o, g2o,
                  idx_r, idx_c, g1a, g1b, g2a, g2b, sga, sgb, swa, swb):
        c = lax.axis_index("c")
        s = lax.axis_index("s")
        w = s * NC + c
        base = w * EPW
        pltpu.sync_copy(rowr.at[w], idx_r)
        pltpu.sync_copy(colr.at[w], idx_c)

        def body(g, carry):
            i0 = 2 * g
            i1 = 2 * g + 1
            off0 = base + i0 * CH
            off1 = base + i1 * CH

            @pl.when(g >= 1)
            def _():
                pltpu.make_async_copy(g1a, g1o.at[pl.ds(off0, CH)], swa).wait()
                pltpu.make_async_copy(g2a, g2o.at[pl.ds(off0, CH)], swa).wait()
            cpa1 = pltpu.async_copy(trow.at[idx_r.at[i0]], g1a, sga)
            cpa2 = pltpu.async_copy(tcol.at[idx_c.at[i0]], g2a, sga)

            @pl.when(g >= 1)
            def _():
                pltpu.make_async_copy(g1b, g1o.at[pl.ds(off1, CH)], swb).wait()
                pltpu.make_async_copy(g2b, g2o.at[pl.ds(off1, CH)], swb).wait()
            cpb1 = pltpu.async_copy(trow.at[idx_r.at[i1]], g1b, sgb)
            cpb2 = pltpu.async_copy(tcol.at[idx_c.at[i1]], g2b, sgb)

            cpa1.wait()
            cpa2.wait()
            pltpu.async_copy(g1a, g1o.at[pl.ds(off0, CH)], swa)
            pltpu.async_copy(g2a, g2o.at[pl.ds(off0, CH)], swa)
            cpb1.wait()
            cpb2.wait()
            pltpu.async_copy(g1b, g1o.at[pl.ds(off1, CH)], swb)
            pltpu.async_copy(g2b, g2o.at[pl.ds(off1, CH)], swb)
            return carry

        lax.fori_loop(0, NG, body, 0)
        pltpu.make_async_copy(g1a, g1o.at[pl.ds(base, CH)], swa).wait()
        pltpu.make_async_copy(g2a, g2o.at[pl.ds(base, CH)], swa).wait()
        pltpu.make_async_copy(g1b, g1o.at[pl.ds(base, CH)], swb).wait()
        pltpu.make_async_copy(g2b, g2o.at[pl.ds(base, CH)], swb).wait()

    @functools.partial(
        pl.kernel,
        out_type=jax.ShapeDtypeStruct((NC * NP, GD), F32),
        mesh=mesh,
        scratch_types=[
            pltpu.VMEM_SHARED((NP, GD), F32),
            pltpu.VMEM((NCH, CH), jnp.int32),
            pltpu.VMEM((CH, GD), F32),
            pltpu.SemaphoreType.DMA,
            pltpu.SemaphoreType.DMA,
        ],
    )
    def sc_count(colr, zz, onesh, outs, shared, idx2d, ones_v, saa, sab):
        c = lax.axis_index("c")
        s = lax.axis_index("s")
        w = s * NC + c
        pltpu.sync_copy(zz.at[pl.ds(s * RPT, RPT)],
                        shared.at[pl.ds(s * RPT, RPT)])
        pltpu.sync_copy(colr.at[w], idx2d)
        pltpu.sync_copy(onesh, ones_v)
        plsc.subcore_barrier()

        def body(g, carry):
            i0 = 2 * g
            i1 = 2 * g + 1

            @pl.when(g >= 1)
            def _():
                pltpu.make_async_copy(ones_v, shared.at[pl.ds(0, CH)],
                                      saa).wait()
            pltpu.async_copy(ones_v, shared.at[idx2d.at[i0]], saa, add=True)

            @pl.when(g >= 1)
            def _():
                pltpu.make_async_copy(ones_v, shared.at[pl.ds(0, CH)],
                                      sab).wait()
            pltpu.async_copy(ones_v, shared.at[idx2d.at[i1]], sab, add=True)
            return carry

        lax.fori_loop(0, NG, body, 0)
        pltpu.make_async_copy(ones_v, shared.at[pl.ds(0, CH)], saa).wait()
        pltpu.make_async_copy(ones_v, shared.at[pl.ds(0, CH)], sab).wait()
        plsc.subcore_barrier()
        pltpu.sync_copy(shared.at[pl.ds(s * RPT, RPT)],
                        outs.at[pl.ds(c * NP + s * RPT, RPT)])

    @functools.partial(
        pl.kernel,
        out_type=jax.ShapeDtypeStruct((NC * NP, GD), F32),
        mesh=mesh,
        scratch_types=[
            pltpu.VMEM_SHARED((NP, GD), F32),
            pltpu.VMEM((NCH, CH), jnp.int32),
            pltpu.VMEM((CH, GD), F32),
            pltpu.VMEM((CH, GD), F32),
            pltpu.SemaphoreType.DMA,
            pltpu.SemaphoreType.DMA,
            pltpu.SemaphoreType.DMA,
            pltpu.SemaphoreType.DMA,
        ],
    )
    def sc_scatter(n1, colr, zz, outs, shared, idx2d,
                   vals_a, vals_b, sva, svb, saa, sab):
        c = lax.axis_index("c")
        s = lax.axis_index("s")
        w = s * NC + c
        base = w * EPW
        pltpu.sync_copy(zz.at[pl.ds(s * RPT, RPT)],
                        shared.at[pl.ds(s * RPT, RPT)])
        pltpu.sync_copy(colr.at[w], idx2d)
        plsc.subcore_barrier()

        def body(g, carry):
            i0 = 2 * g
            i1 = 2 * g + 1

            @pl.when(g >= 1)
            def _():
                pltpu.make_async_copy(vals_a, shared.at[pl.ds(0, CH)],
                                      saa).wait()
            cpa = pltpu.async_copy(n1.at[pl.ds(base + i0 * CH, CH)],
                                   vals_a, sva)

            @pl.when(g >= 1)
            def _():
                pltpu.make_async_copy(vals_b, shared.at[pl.ds(0, CH)],
                                      sab).wait()
            cpb = pltpu.async_copy(n1.at[pl.ds(base + i1 * CH, CH)],
                                   vals_b, svb)

            cpa.wait()
            pltpu.async_copy(vals_a, shared.at[idx2d.at[i0]], saa, add=True)
            cpb.wait()
            pltpu.async_copy(vals_b, shared.at[idx2d.at[i1]], sab, add=True)
            return carry

        lax.fori_loop(0, NG, body, 0)
        pltpu.make_async_copy(vals_a, shared.at[pl.ds(0, CH)], saa).wait()
        pltpu.make_async_copy(vals_b, shared.at[pl.ds(0, CH)], sab).wait()
        plsc.subcore_barrier()
        pltpu.sync_copy(shared.at[pl.ds(s * RPT, RPT)],
                        outs.at[pl.ds(c * NP + s * RPT, RPT)])

    return sc_gather, sc_count, sc_scatter


def _dot(a, b):
    return jnp.dot(a, b, preferred_element_type=F32)


def _wspec(r, c=GD):
    return pl.BlockSpec((r, c), lambda i: (0, 0))


def _proj_body(x_ref, wr, wb, wc, trow_o, tcol_o):
    x = x_ref[...]
    trow_o[...] = jnp.concatenate([_dot(x, wr[...]), _dot(x, wb[...])], axis=1)
    tcol_o[...] = _dot(x, wc[...])


_tc_proj = pl.pallas_call(
    _proj_body,
    grid=(GN,),
    in_specs=[pl.BlockSpec((BN, GD), lambda i: (i, 0)),
              _wspec(GD), _wspec(GD), _wspec(GD)],
    out_specs=[pl.BlockSpec((BN, 2 * GD), lambda i: (i, 0)),
               pl.BlockSpec((BN, GD), lambda i: (i, 0))],
    out_shape=[jax.ShapeDtypeStruct((N, 2 * GD), F32),
               jax.ShapeDtypeStruct((N, GD), F32)],
)


def _make_edge(ewd, write_e):
    def body(g1, g2, ew, wew, w2e, wn1e, wn2, be1, b2e, bn1, bn2, *outs):
        g1v = g1[...]
        h1 = jnp.maximum(
            g1v[:, :GD] + g2[...] + _dot(ew[...], wew[...]) + be1[...], 0.0)
        e = _dot(h1, w2e[...]) + b2e[...]
        h2 = jnp.maximum(g1v[:, GD:] + _dot(e, wn1e[...]) + bn1[...], 0.0)
        n1v = _dot(h2, wn2[...]) + bn2[...]
        if write_e:
            outs[0][...] = e
            outs[1][...] = n1v
        else:
            outs[0][...] = n1v

    out_specs = [pl.BlockSpec((BE, GD), lambda i: (i, 0))]
    out_shape = [jax.ShapeDtypeStruct((E, GD), F32)]
    if write_e:
        out_specs = out_specs * 2
        out_shape = out_shape * 2
    return pl.pallas_call(
        body,
        grid=(GE,),
        in_specs=[pl.BlockSpec((BE, 2 * GD), lambda i: (i, 0)),
                  pl.BlockSpec((BE, GD), lambda i: (i, 0)),
                  pl.BlockSpec((BE, ewd), lambda i: (i, 0)),
                  _wspec(ewd), _wspec(GD), _wspec(GD), _wspec(GD),
                  _wspec(1), _wspec(1), _wspec(1), _wspec(1)],
        out_specs=out_specs,
        out_shape=out_shape,
    )


_tc_edge1 = _make_edge(EID, True)
_tc_edge2 = _make_edge(GD, True)
_tc_edge3 = _make_edge(GD, False)


def _make_node(first, last):
    def body(*refs):
        it = iter(refs)
        p0, p1 = next(it), next(it)
        if first:
            c0, c1 = next(it), next(it)
        else:
            inv_in = next(it)
        x_ref, wmx, wma, wm2, ubm, bm2 = (next(it) for _ in range(6))
        if not last:
            wr2, wb2, wc2 = next(it), next(it), next(it)
            xn_o, trow_o, tcol_o = next(it), next(it), next(it)
        mp_o = next(it)
        if first:
            inv_o = next(it)

        i = pl.program_id(0)
        s = p0[0] + p1[0]
        if first:
            cnt = c0[0][:, :1] + c1[0][:, :1]
            invb = 1.0 / jnp.maximum(cnt, 1.0)
            inv_o[...] = jnp.broadcast_to(invb, (BN, GD))
            agg = s * invb
        else:
            agg = s * inv_in[...]
        x = x_ref[...]
        h = jnp.maximum(_dot(x, wmx[...]) + _dot(agg, wma[...]) + ubm[...], 0.0)
        xn = _dot(h, wm2[...]) + bm2[...]
        if not last:
            xn_o[...] = xn
            trow_o[...] = jnp.concatenate(
                [_dot(xn, wr2[...]), _dot(xn, wb2[...])], axis=1)
            tcol_o[...] = _dot(xn, wc2[...])

        @pl.when(i == 0)
        def _():
            mp_o[...] = jnp.zeros((8, GD), F32)

        mp_o[0:1, :] = mp_o[0:1, :] + jnp.sum(xn, axis=0, keepdims=True)

    p_spec = [pl.BlockSpec((1, BN, GD), lambda i: (0, i, 0)),
              pl.BlockSpec((1, BN, GD), lambda i: (1, i, 0))]
    in_specs = list(p_spec)
    if first:
        in_specs += [pl.BlockSpec((1, BN, GD), lambda i: (0, i, 0)),
                     pl.BlockSpec((1, BN, GD), lambda i: (1, i, 0))]
    else:
        in_specs += [pl.BlockSpec((BN, GD), lambda i: (i, 0))]
    in_specs += [pl.BlockSpec((BN, GD), lambda i: (i, 0)),
                 _wspec(GD), _wspec(GD), _wspec(GD), _wspec(1), _wspec(1)]
    if not last:
        in_specs += [_wspec(GD), _wspec(GD), _wspec(GD)]

    out_specs, out_shape = [], []
    if not last:
        out_specs += [pl.BlockSpec((BN, GD), lambda i: (i, 0)),
                      pl.BlockSpec((BN, 2 * GD), lambda i: (i, 0)),
                      pl.BlockSpec((BN, GD), lambda i: (i, 0))]
        out_shape += [jax.ShapeDtypeStruct((N, GD), F32),
                      jax.ShapeDtypeStruct((N, 2 * GD), F32),
                      jax.ShapeDtypeStruct((N, GD), F32)]
    out_specs += [pl.BlockSpec((8, GD), lambda i: (0, 0))]
    out_shape += [jax.ShapeDtypeStruct((8, GD), F32)]
    if first:
        out_specs += [pl.BlockSpec((BN, GD), lambda i: (i, 0))]
        out_shape += [jax.ShapeDtypeStruct((N, GD), F32)]
    return pl.pallas_call(body, grid=(GN,), in_specs=in_specs,
                          out_specs=out_specs, out_shape=out_shape)


_tc_node1 = _make_node(True, False)
_tc_node2 = _make_node(False, False)
_tc_node3 = _make_node(False, True)


def _make_glob(last):
    def body(mp, u, wgu, wgm, bg1, wg2, bg2, *rest):
        m = jnp.sum(mp[...], axis=0, keepdims=True) * (1.0 / N)
        m8 = jnp.broadcast_to(m, (8, GD))
        h = jnp.maximum(_dot(u[...], wgu[...]) + _dot(m8, wgm[...]) + bg1[...],
                        0.0)
        un = _dot(h, wg2[...]) + bg2[...]
        if last:
            wlin, blin, out_o = rest
            out_o[...] = (_dot(un, wlin[...]) + blin[...])[0:1, :]
        else:
            wue, b1e, wum, bm1, un_o, be1_o, ubm_o = rest
            un_o[...] = un
            be1_o[...] = (_dot(un, wue[...]) + b1e[...])[0:1, :]
            ubm_o[...] = (_dot(un, wum[...]) + bm1[...])[0:1, :]

    in_specs = [_wspec(8), _wspec(8), _wspec(GD), _wspec(GD), _wspec(1),
                _wspec(GD), _wspec(1)]
    if last:
        in_specs += [_wspec(GD), _wspec(1)]
        out_specs = [_wspec(1)]
        out_shape = [jax.ShapeDtypeStruct((1, GD), F32)]
    else:
        in_specs += [_wspec(GD), _wspec(1), _wspec(GD), _wspec(1)]
        out_specs = [_wspec(8), _wspec(1), _wspec(1)]
        out_shape = [jax.ShapeDtypeStruct((8, GD), F32),
                     jax.ShapeDtypeStruct((1, GD), F32),
                     jax.ShapeDtypeStruct((1, GD), F32)]
    return pl.pallas_call(body, grid=(1,), in_specs=in_specs,
                          out_specs=out_specs, out_shape=out_shape)


_tc_glob_mid = _make_glob(False)
_tc_glob_last = _make_glob(True)


def _split(p, ni, ewd, has_u):
    (w1e, b1e, w2e, b2e) = p['edge']
    (wn1, bn1, wn2, bn2) = p['nm1']
    (wm1, bm1, wm2, bm2) = p['nm2']
    (wg1, bg1, wg2, bg2) = p['glob']
    r = lambda b: b.reshape(1, GD)
    d = {
        'wr': w1e[:, :ni].T, 'wc': w1e[:, ni:2 * ni].T,
        'wew': w1e[:, 2 * ni:2 * ni + ewd].T,
        'b1e': r(b1e), 'w2e': w2e.T, 'b2e': r(b2e),
        'wbx': wn1[:, :ni].T, 'wn1e': wn1[:, ni:].T, 'bn1': r(bn1),
        'wn2': wn2.T, 'bn2': r(bn2),
        'wmx': wm1[:, :ni].T, 'wma': wm1[:, ni:ni + GD].T, 'bm1': r(bm1),
        'wm2': wm2.T, 'bm2': r(bm2),
        'wgu': wg1[:, :GD].T, 'wgm': wg1[:, GD:].T, 'bg1': r(bg1),
        'wg2': wg2.T, 'bg2': r(bg2),
    }
    if has_u:
        d['wue'] = w1e[:, 2 * ni + ewd:].T
        d['wum'] = wm1[:, ni + GD:].T
    return d


def kernel(x, edge_index, edge_w, params):
    _sc_gather, _sc_count, _sc_scatter = _sc_kernels()
    row, col = edge_index[0], edge_index[1]
    q1 = _split(params['conv1'], GD, EID, False)
    q2 = _split(params['conv2'], GD, GD, True)
    q3 = _split(params['conv3'], GD, GD, True)
    wlin, blin = params['lin'][0].T, params['lin'][1].reshape(1, GD)

    u0 = jax.random.normal(jax.random.key(42), (1, GD), F32) * 0.01
    u0p = jnp.concatenate([u0, jnp.zeros((7, GD), F32)], axis=0)
    zz = jnp.zeros((NP, GD), F32)
    onesh = jnp.ones((CH, GD), F32)

    rowr = row.reshape(NW, NCH, CH)
    colr = col.reshape(NW, NCH, CH)
    trow, tcol = _tc_proj(x, q1['wr'], q1['wbx'], q1['wc'])
    g1, g2 = _sc_gather(trow, tcol, rowr, colr)
    e1, n1 = _tc_edge1(g1, g2, edge_w, q1['wew'], q1['w2e'], q1['wn1e'],
                       q1['wn2'], q1['b1e'], q1['b2e'], q1['bn1'], q1['bn2'])
    pc = _sc_count(colr, zz, onesh)
    ps = _sc_scatter(n1, colr, zz)
    xn1, trow, tcol, mp, inv = _tc_node1(
        ps.reshape(2, NP, GD), ps.reshape(2, NP, GD),
        pc.reshape(2, NP, GD), pc.reshape(2, NP, GD),
        x, q1['wmx'], q1['wma'], q1['wm2'], q1['bm1'], q1['bm2'],
        q2['wr'], q2['wbx'], q2['wc'])
    u1, be1_2, ubm2 = _tc_glob_mid(mp, u0p, q1['wgu'], q1['wgm'], q1['bg1'],
                                   q1['wg2'], q1['bg2'],
                                   q2['wue'], q2['b1e'], q2['wum'], q2['bm1'])

    g1, g2 = _sc_gather(trow, tcol, rowr, colr)
    e2, n2 = _tc_edge2(g1, g2, e1, q2['wew'], q2['w2e'], q2['wn1e'],
                       q2['wn2'], be1_2, q2['b2e'], q2['bn1'], q2['bn2'])
    ps = _sc_scatter(n2, colr, zz)
    xn2, trow, tcol, mp = _tc_node2(
        ps.reshape(2, NP, GD), ps.reshape(2, NP, GD), inv,
        xn1, q2['wmx'], q2['wma'], q2['wm2'], ubm2, q2['bm2'],
        q3['wr'], q3['wbx'], q3['wc'])
    u2, be1_3, ubm3 = _tc_glob_mid(mp, u1, q2['wgu'], q2['wgm'], q2['bg1'],
                                   q2['wg2'], q2['bg2'],
                                   q3['wue'], q3['b1e'], q3['wum'], q3['bm1'])

    g1, g2 = _sc_gather(trow, tcol, rowr, colr)
    (n3,) = _tc_edge3(g1, g2, e2, q3['wew'], q3['w2e'], q3['wn1e'],
                      q3['wn2'], be1_3, q3['b2e'], q3['bn1'], q3['bn2'])
    ps = _sc_scatter(n3, colr, zz)
    (mp,) = _tc_node3(ps.reshape(2, NP, GD), ps.reshape(2, NP, GD), inv,
                      xn2, q3['wmx'], q3['wma'], q3['wm2'], ubm3, q3['bm2'])
    (out,) = _tc_glob_last(mp, u2, q3['wgu'], q3['wgm'], q3['bg1'],
                           q3['wg2'], q3['bg2'], wlin, blin)
    return out


# speedup vs baseline: 2.3469x; 1.1564x over previous
import functools
import jax
import jax.numpy as jnp
from jax import lax
from jax.experimental import pallas as pl
from jax.experimental.pallas import tpu as pltpu
from jax.experimental.pallas import tpu_sc as plsc

N = 10000
E = 320000
GD = 128
EID = 16
F32 = jnp.float32

NC, NS = 2, 16
NW = NC * NS
EPW = E // NW
CH = 40
NCH = EPW // CH
NG = NCH // 2
NP = 10240
RPT = NP // NS

BE = 512
GE = E // BE
BN = 2000
GN = N // BN

@functools.lru_cache(maxsize=None)
def _sc_kernels():
    mesh = plsc.VectorSubcoreMesh(core_axis_name="c", subcore_axis_name="s",
                                  num_cores=NC, num_subcores=NS)

    @functools.partial(
        pl.kernel,
        out_type=[jax.ShapeDtypeStruct((E, GD), F32),
                  jax.ShapeDtypeStruct((E, GD), F32)],
        mesh=mesh,
        scratch_types=[
            pltpu.VMEM((NCH, CH), jnp.int32),
            pltpu.VMEM((NCH, CH), jnp.int32),
            pltpu.VMEM((CH, GD), F32),
            pltpu.VMEM((CH, GD), F32),
            pltpu.VMEM((CH, GD), F32),
            pltpu.VMEM((CH, GD), F32),
            pltpu.SemaphoreType.DMA,
            pltpu.SemaphoreType.DMA,
            pltpu.SemaphoreType.DMA,
            pltpu.SemaphoreType.DMA,
        ],
    )
    def sc_gather(trow, tcol, rowr, colr, g1o, g2o,
                  idx_r, idx_c, g1a, g1b, g2a, g2b, sga, sgb, swa, swb):
        c = lax.axis_index("c")
        s = lax.axis_index("s")
        w = s * NC + c
        base = w * EPW
        pltpu.sync_copy(rowr.at[w], idx_r)
        pltpu.sync_copy(colr.at[w], idx_c)

        def body(g, carry):
            i0 = 2 * g
            i1 = 2 * g + 1
            off0 = base + i0 * CH
            off1 = base + i1 * CH

            @pl.when(g >= 1)
            def _():
                pltpu.make_async_copy(g1a, g1o.at[pl.ds(off0, CH)], swa).wait()
                pltpu.make_async_copy(g2a, g2o.at[pl.ds(off0, CH)], swa).wait()
            cpa1 = pltpu.async_copy(trow.at[idx_r.at[i0]], g1a, sga)
            cpa2 = pltpu.async_copy(tcol.at[idx_c.at[i0]], g2a, sga)

            @pl.when(g >= 1)
            def _():
                pltpu.make_async_copy(g1b, g1o.at[pl.ds(off1, CH)], swb).wait()
                pltpu.make_async_copy(g2b, g2o.at[pl.ds(off1, CH)], swb).wait()
            cpb1 = pltpu.async_copy(trow.at[idx_r.at[i1]], g1b, sgb)
            cpb2 = pltpu.async_copy(tcol.at[idx_c.at[i1]], g2b, sgb)

            cpa1.wait()
            cpa2.wait()
            pltpu.async_copy(g1a, g1o.at[pl.ds(off0, CH)], swa)
            pltpu.async_copy(g2a, g2o.at[pl.ds(off0, CH)], swa)
            cpb1.wait()
            cpb2.wait()
            pltpu.async_copy(g1b, g1o.at[pl.ds(off1, CH)], swb)
            pltpu.async_copy(g2b, g2o.at[pl.ds(off1, CH)], swb)
            return carry

        lax.fori_loop(0, NG, body, 0)
        pltpu.make_async_copy(g1a, g1o.at[pl.ds(base, CH)], swa).wait()
        pltpu.make_async_copy(g2a, g2o.at[pl.ds(base, CH)], swa).wait()
        pltpu.make_async_copy(g1b, g1o.at[pl.ds(base, CH)], swb).wait()
        pltpu.make_async_copy(g2b, g2o.at[pl.ds(base, CH)], swb).wait()

    @functools.partial(
        pl.kernel,
        out_type=jax.ShapeDtypeStruct((NC * NP, GD), F32),
        mesh=mesh,
        scratch_types=[
            pltpu.VMEM_SHARED((NP, GD), F32),
            pltpu.VMEM((NCH, CH), jnp.int32),
            pltpu.VMEM((CH, GD), F32),
            pltpu.SemaphoreType.DMA,
            pltpu.SemaphoreType.DMA,
        ],
    )
    def sc_count(colr, zz, onesh, outs, shared, idx2d, ones_v, saa, sab):
        c = lax.axis_index("c")
        s = lax.axis_index("s")
        w = s * NC + c
        pltpu.sync_copy(zz.at[pl.ds(s * RPT, RPT)],
                        shared.at[pl.ds(s * RPT, RPT)])
        pltpu.sync_copy(colr.at[w], idx2d)
        pltpu.sync_copy(onesh, ones_v)
        plsc.subcore_barrier()

        def body(g, carry):
            i0 = 2 * g
            i1 = 2 * g + 1

            @pl.when(g >= 1)
            def _():
                pltpu.make_async_copy(ones_v, shared.at[pl.ds(0, CH)],
                                      saa).wait()
            pltpu.async_copy(ones_v, shared.at[idx2d.at[i0]], saa, add=True)

            @pl.when(g >= 1)
            def _():
                pltpu.make_async_copy(ones_v, shared.at[pl.ds(0, CH)],
                                      sab).wait()
            pltpu.async_copy(ones_v, shared.at[idx2d.at[i1]], sab, add=True)
            return carry

        lax.fori_loop(0, NG, body, 0)
        pltpu.make_async_copy(ones_v, shared.at[pl.ds(0, CH)], saa).wait()
        pltpu.make_async_copy(ones_v, shared.at[pl.ds(0, CH)], sab).wait()
        plsc.subcore_barrier()
        pltpu.sync_copy(shared.at[pl.ds(s * RPT, RPT)],
                        outs.at[pl.ds(c * NP + s * RPT, RPT)])

    @functools.partial(
        pl.kernel,
        out_type=jax.ShapeDtypeStruct((NC * NP, GD), F32),
        mesh=mesh,
        scratch_types=[
            pltpu.VMEM_SHARED((NP, GD), F32),
            pltpu.VMEM((NCH, CH), jnp.int32),
            pltpu.VMEM((CH, GD), F32),
            pltpu.VMEM((CH, GD), F32),
            pltpu.SemaphoreType.DMA,
            pltpu.SemaphoreType.DMA,
            pltpu.SemaphoreType.DMA,
            pltpu.SemaphoreType.DMA,
        ],
    )
    def sc_scatter(n1, colr, zz, outs, shared, idx2d,
                   vals_a, vals_b, sva, svb, saa, sab):
        c = lax.axis_index("c")
        s = lax.axis_index("s")
        w = s * NC + c
        base = w * EPW
        pltpu.sync_copy(zz.at[pl.ds(s * RPT, RPT)],
                        shared.at[pl.ds(s * RPT, RPT)])
        pltpu.sync_copy(colr.at[w], idx2d)
        plsc.subcore_barrier()

        def body(g, carry):
            i0 = 2 * g
            i1 = 2 * g + 1

            @pl.when(g >= 1)
            def _():
                pltpu.make_async_copy(vals_a, shared.at[pl.ds(0, CH)],
                                      saa).wait()
            cpa = pltpu.async_copy(n1.at[pl.ds(base + i0 * CH, CH)],
                                   vals_a, sva)

            @pl.when(g >= 1)
            def _():
                pltpu.make_async_copy(vals_b, shared.at[pl.ds(0, CH)],
                                      sab).wait()
            cpb = pltpu.async_copy(n1.at[pl.ds(base + i1 * CH, CH)],
                                   vals_b, svb)

            cpa.wait()
            pltpu.async_copy(vals_a, shared.at[idx2d.at[i0]], saa, add=True)
            cpb.wait()
            pltpu.async_copy(vals_b, shared.at[idx2d.at[i1]], sab, add=True)
            return carry

        lax.fori_loop(0, NG, body, 0)
        pltpu.make_async_copy(vals_a, shared.at[pl.ds(0, CH)], saa).wait()
        pltpu.make_async_copy(vals_b, shared.at[pl.ds(0, CH)], sab).wait()
        plsc.subcore_barrier()
        pltpu.sync_copy(shared.at[pl.ds(s * RPT, RPT)],
                        outs.at[pl.ds(c * NP + s * RPT, RPT)])

    return sc_gather, sc_count, sc_scatter


def _dot(a, b):
    return jnp.dot(a, b, preferred_element_type=F32)


def _pack2(a, b):
    au = jax.lax.bitcast_convert_type(a.astype(jnp.bfloat16), jnp.uint16)
    bu = jax.lax.bitcast_convert_type(b.astype(jnp.bfloat16), jnp.uint16)
    w = au.astype(jnp.uint32) | (bu.astype(jnp.uint32) << 16)
    return jax.lax.bitcast_convert_type(w, F32)


def _unpack2(p):
    w = jax.lax.bitcast_convert_type(p, jnp.uint32)
    a = jax.lax.bitcast_convert_type(w << 16, F32)
    b = jax.lax.bitcast_convert_type(w & jnp.uint32(0xFFFF0000), F32)
    return a, b


def _wspec(r, c=GD):
    return pl.BlockSpec((r, c), lambda i: (0, 0))


def _proj_body(x_ref, wr, wb, wc, trow_o, tcol_o):
    x = x_ref[...]
    trow_o[...] = _pack2(_dot(x, wr[...]), _dot(x, wb[...]))
    tcol_o[...] = _dot(x, wc[...])


_tc_proj = pl.pallas_call(
    _proj_body,
    grid=(GN,),
    in_specs=[pl.BlockSpec((BN, GD), lambda i: (i, 0)),
              _wspec(GD), _wspec(GD), _wspec(GD)],
    out_specs=[pl.BlockSpec((BN, GD), lambda i: (i, 0)),
               pl.BlockSpec((BN, GD), lambda i: (i, 0))],
    out_shape=[jax.ShapeDtypeStruct((N, GD), F32),
               jax.ShapeDtypeStruct((N, GD), F32)],
)


def _make_edge(ewd, write_e):
    def body(g1, g2, ew, wew, w2e, wn1e, wn2, be1, b2e, bn1, bn2, *outs):
        ar, br = _unpack2(g1[...])
        h1 = jnp.maximum(
            ar + g2[...] + _dot(ew[...].astype(F32), wew[...]) + be1[...], 0.0)
        e = _dot(h1, w2e[...]) + b2e[...]
        h2 = jnp.maximum(br + _dot(e, wn1e[...]) + bn1[...], 0.0)
        n1v = _dot(h2, wn2[...]) + bn2[...]
        if write_e:
            outs[0][...] = e.astype(jnp.bfloat16)
            outs[1][...] = n1v
        else:
            outs[0][...] = n1v

    out_specs = [pl.BlockSpec((BE, GD), lambda i: (i, 0))]
    out_shape = [jax.ShapeDtypeStruct((E, GD), F32)]
    if write_e:
        out_specs = out_specs * 2
        out_shape = [jax.ShapeDtypeStruct((E, GD), jnp.bfloat16)] + out_shape
    return pl.pallas_call(
        body,
        grid=(GE,),
        in_specs=[pl.BlockSpec((BE, GD), lambda i: (i, 0)),
                  pl.BlockSpec((BE, GD), lambda i: (i, 0)),
                  pl.BlockSpec((BE, ewd), lambda i: (i, 0)),
                  _wspec(ewd), _wspec(GD), _wspec(GD), _wspec(GD),
                  _wspec(1), _wspec(1), _wspec(1), _wspec(1)],
        out_specs=out_specs,
        out_shape=out_shape,
    )


_tc_edge1 = _make_edge(EID, True)
_tc_edge2 = _make_edge(GD, True)
_tc_edge3 = _make_edge(GD, False)


def _make_node(first, last):
    def body(*refs):
        it = iter(refs)
        p0, p1 = next(it), next(it)
        if first:
            c0, c1 = next(it), next(it)
        else:
            inv_in = next(it)
        x_ref, wmx, wma, wm2, ubm, bm2 = (next(it) for _ in range(6))
        if not last:
            wr2, wb2, wc2 = next(it), next(it), next(it)
            xn_o, trow_o, tcol_o = next(it), next(it), next(it)
        mp_o = next(it)
        if first:
            inv_o = next(it)

        i = pl.program_id(0)
        s = p0[0] + p1[0]
        if first:
            cnt = c0[0][:, :1] + c1[0][:, :1]
            invb = 1.0 / jnp.maximum(cnt, 1.0)
            inv_o[...] = jnp.broadcast_to(invb, (BN, GD))
            agg = s * invb
        else:
            agg = s * inv_in[...]
        x = x_ref[...]
        h = jnp.maximum(_dot(x, wmx[...]) + _dot(agg, wma[...]) + ubm[...], 0.0)
        xn = _dot(h, wm2[...]) + bm2[...]
        if not last:
            xn_o[...] = xn
            trow_o[...] = _pack2(_dot(xn, wr2[...]), _dot(xn, wb2[...]))
            tcol_o[...] = _dot(xn, wc2[...])

        @pl.when(i == 0)
        def _():
            mp_o[...] = jnp.zeros((8, GD), F32)

        mp_o[0:1, :] = mp_o[0:1, :] + jnp.sum(xn, axis=0, keepdims=True)

    p_spec = [pl.BlockSpec((1, BN, GD), lambda i: (0, i, 0)),
              pl.BlockSpec((1, BN, GD), lambda i: (1, i, 0))]
    in_specs = list(p_spec)
    if first:
        in_specs += [pl.BlockSpec((1, BN, GD), lambda i: (0, i, 0)),
                     pl.BlockSpec((1, BN, GD), lambda i: (1, i, 0))]
    else:
        in_specs += [pl.BlockSpec((BN, GD), lambda i: (i, 0))]
    in_specs += [pl.BlockSpec((BN, GD), lambda i: (i, 0)),
                 _wspec(GD), _wspec(GD), _wspec(GD), _wspec(1), _wspec(1)]
    if not last:
        in_specs += [_wspec(GD), _wspec(GD), _wspec(GD)]

    out_specs, out_shape = [], []
    if not last:
        out_specs += [pl.BlockSpec((BN, GD), lambda i: (i, 0)),
                      pl.BlockSpec((BN, GD), lambda i: (i, 0)),
                      pl.BlockSpec((BN, GD), lambda i: (i, 0))]
        out_shape += [jax.ShapeDtypeStruct((N, GD), F32),
                      jax.ShapeDtypeStruct((N, GD), F32),
                      jax.ShapeDtypeStruct((N, GD), F32)]
    out_specs += [pl.BlockSpec((8, GD), lambda i: (0, 0))]
    out_shape += [jax.ShapeDtypeStruct((8, GD), F32)]
    if first:
        out_specs += [pl.BlockSpec((BN, GD), lambda i: (i, 0))]
        out_shape += [jax.ShapeDtypeStruct((N, GD), F32)]
    return pl.pallas_call(body, grid=(GN,), in_specs=in_specs,
                          out_specs=out_specs, out_shape=out_shape)


_tc_node1 = _make_node(True, False)
_tc_node2 = _make_node(False, False)
_tc_node3 = _make_node(False, True)


def _make_glob(last):
    def body(mp, u, wgu, wgm, bg1, wg2, bg2, *rest):
        m = jnp.sum(mp[...], axis=0, keepdims=True) * (1.0 / N)
        m8 = jnp.broadcast_to(m, (8, GD))
        h = jnp.maximum(_dot(u[...], wgu[...]) + _dot(m8, wgm[...]) + bg1[...],
                        0.0)
        un = _dot(h, wg2[...]) + bg2[...]
        if last:
            wlin, blin, out_o = rest
            out_o[...] = (_dot(un, wlin[...]) + blin[...])[0:1, :]
        else:
            wue, b1e, wum, bm1, un_o, be1_o, ubm_o = rest
            un_o[...] = un
            be1_o[...] = (_dot(un, wue[...]) + b1e[...])[0:1, :]
            ubm_o[...] = (_dot(un, wum[...]) + bm1[...])[0:1, :]

    in_specs = [_wspec(8), _wspec(8), _wspec(GD), _wspec(GD), _wspec(1),
                _wspec(GD), _wspec(1)]
    if last:
        in_specs += [_wspec(GD), _wspec(1)]
        out_specs = [_wspec(1)]
        out_shape = [jax.ShapeDtypeStruct((1, GD), F32)]
    else:
        in_specs += [_wspec(GD), _wspec(1), _wspec(GD), _wspec(1)]
        out_specs = [_wspec(8), _wspec(1), _wspec(1)]
        out_shape = [jax.ShapeDtypeStruct((8, GD), F32),
                     jax.ShapeDtypeStruct((1, GD), F32),
                     jax.ShapeDtypeStruct((1, GD), F32)]
    return pl.pallas_call(body, grid=(1,), in_specs=in_specs,
                          out_specs=out_specs, out_shape=out_shape)


_tc_glob_mid = _make_glob(False)
_tc_glob_last = _make_glob(True)


def _split(p, ni, ewd, has_u):
    (w1e, b1e, w2e, b2e) = p['edge']
    (wn1, bn1, wn2, bn2) = p['nm1']
    (wm1, bm1, wm2, bm2) = p['nm2']
    (wg1, bg1, wg2, bg2) = p['glob']
    r = lambda b: b.reshape(1, GD)
    d = {
        'wr': w1e[:, :ni].T, 'wc': w1e[:, ni:2 * ni].T,
        'wew': w1e[:, 2 * ni:2 * ni + ewd].T,
        'b1e': r(b1e), 'w2e': w2e.T, 'b2e': r(b2e),
        'wbx': wn1[:, :ni].T, 'wn1e': wn1[:, ni:].T, 'bn1': r(bn1),
        'wn2': wn2.T, 'bn2': r(bn2),
        'wmx': wm1[:, :ni].T, 'wma': wm1[:, ni:ni + GD].T, 'bm1': r(bm1),
        'wm2': wm2.T, 'bm2': r(bm2),
        'wgu': wg1[:, :GD].T, 'wgm': wg1[:, GD:].T, 'bg1': r(bg1),
        'wg2': wg2.T, 'bg2': r(bg2),
    }
    if has_u:
        d['wue'] = w1e[:, 2 * ni + ewd:].T
        d['wum'] = wm1[:, ni + GD:].T
    return d


def kernel(x, edge_index, edge_w, params):
    _sc_gather, _sc_count, _sc_scatter = _sc_kernels()
    row, col = edge_index[0], edge_index[1]
    q1 = _split(params['conv1'], GD, EID, False)
    q2 = _split(params['conv2'], GD, GD, True)
    q3 = _split(params['conv3'], GD, GD, True)
    wlin, blin = params['lin'][0].T, params['lin'][1].reshape(1, GD)

    u0 = jax.random.normal(jax.random.key(42), (1, GD), F32) * 0.01
    u0p = jnp.concatenate([u0, jnp.zeros((7, GD), F32)], axis=0)
    zz = jnp.zeros((NP, GD), F32)
    onesh = jnp.ones((CH, GD), F32)

    rowr = row.reshape(NW, NCH, CH)
    colr = col.reshape(NW, NCH, CH)
    trow, tcol = _tc_proj(x, q1['wr'], q1['wbx'], q1['wc'])
    g1, g2 = _sc_gather(trow, tcol, rowr, colr)
    e1, n1 = _tc_edge1(g1, g2, edge_w, q1['wew'], q1['w2e'], q1['wn1e'],
                       q1['wn2'], q1['b1e'], q1['b2e'], q1['bn1'], q1['bn2'])
    pc = _sc_count(colr, zz, onesh)
    ps = _sc_scatter(n1, colr, zz)
    xn1, trow, tcol, mp, inv = _tc_node1(
        ps.reshape(2, NP, GD), ps.reshape(2, NP, GD),
        pc.reshape(2, NP, GD), pc.reshape(2, NP, GD),
        x, q1['wmx'], q1['wma'], q1['wm2'], q1['bm1'], q1['bm2'],
        q2['wr'], q2['wbx'], q2['wc'])
    u1, be1_2, ubm2 = _tc_glob_mid(mp, u0p, q1['wgu'], q1['wgm'], q1['bg1'],
                                   q1['wg2'], q1['bg2'],
                                   q2['wue'], q2['b1e'], q2['wum'], q2['bm1'])

    g1, g2 = _sc_gather(trow, tcol, rowr, colr)
    e2, n2 = _tc_edge2(g1, g2, e1, q2['wew'], q2['w2e'], q2['wn1e'],
                       q2['wn2'], be1_2, q2['b2e'], q2['bn1'], q2['bn2'])
    ps = _sc_scatter(n2, colr, zz)
    xn2, trow, tcol, mp = _tc_node2(
        ps.reshape(2, NP, GD), ps.reshape(2, NP, GD), inv,
        xn1, q2['wmx'], q2['wma'], q2['wm2'], ubm2, q2['bm2'],
        q3['wr'], q3['wbx'], q3['wc'])
    u2, be1_3, ubm3 = _tc_glob_mid(mp, u1, q2['wgu'], q2['wgm'], q2['bg1'],
                                   q2['wg2'], q2['bg2'],
                                   q3['wue'], q3['b1e'], q3['wum'], q3['bm1'])

    g1, g2 = _sc_gather(trow, tcol, rowr, colr)
    (n3,) = _tc_edge3(g1, g2, e2, q3['wew'], q3['w2e'], q3['wn1e'],
                      q3['wn2'], be1_3, q3['b2e'], q3['bn1'], q3['bn2'])
    ps = _sc_scatter(n3, colr, zz)
    (mp,) = _tc_node3(ps.reshape(2, NP, GD), ps.reshape(2, NP, GD), inv,
                      xn2, q3['wmx'], q3['wma'], q3['wm2'], ubm3, q3['bm2'])
    (out,) = _tc_glob_last(mp, u2, q3['wgu'], q3['wgm'], q3['bg1'],
                           q3['wg2'], q3['bg2'], wlin, blin)
    return out


# speedup vs baseline: 2.7162x; 1.1574x over previous
import functools
import jax
import jax.numpy as jnp
from jax import lax
from jax.experimental import pallas as pl
from jax.experimental.pallas import tpu as pltpu
from jax.experimental.pallas import tpu_sc as plsc

N = 10000
E = 320000
GD = 128
EID = 16
F32 = jnp.float32

NC, NS = 2, 16
NW = NC * NS
EPW = E // NW
CH = 40
NCH = EPW // CH
NG = NCH // 2
EA = 163840
EB = E - EA
EPWA, EPWB = EA // NW, EB // NW
NCHA, NCHB = EPWA // CH, EPWB // CH
NGA, NGB = NCHA // 2, NCHB // 2
NP = 10240
RPT = NP // NS

BE = 512
GE = E // BE
BN = 2000
GN = N // BN

@functools.lru_cache(maxsize=None)
def _sc_kernels():
    mesh = plsc.VectorSubcoreMesh(core_axis_name="c", subcore_axis_name="s",
                                  num_cores=NC, num_subcores=NS)

    def make_gather(ee, epw, nch, ng):
        @functools.partial(
            pl.kernel,
            out_type=[jax.ShapeDtypeStruct((ee, GD), F32),
                      jax.ShapeDtypeStruct((ee, GD), F32)],
            mesh=mesh,
            scratch_types=[
                pltpu.VMEM((nch, CH), jnp.int32),
                pltpu.VMEM((nch, CH), jnp.int32),
                pltpu.VMEM((CH, GD), F32),
                pltpu.VMEM((CH, GD), F32),
                pltpu.VMEM((CH, GD), F32),
                pltpu.VMEM((CH, GD), F32),
                pltpu.SemaphoreType.DMA,
                pltpu.SemaphoreType.DMA,
                pltpu.SemaphoreType.DMA,
                pltpu.SemaphoreType.DMA,
            ],
        )
        def sc_gather(trow, tcol, rowr, colr, g1o, g2o,
                      idx_r, idx_c, g1a, g1b, g2a, g2b, sga, sgb, swa, swb):
            c = lax.axis_index("c")
            s = lax.axis_index("s")
            w = s * NC + c
            base = w * epw
            pltpu.sync_copy(rowr.at[w], idx_r)
            pltpu.sync_copy(colr.at[w], idx_c)

            def body(g, carry):
                i0 = 2 * g
                i1 = 2 * g + 1
                off0 = base + i0 * CH
                off1 = base + i1 * CH

                @pl.when(g >= 1)
                def _():
                    pltpu.make_async_copy(g1a, g1o.at[pl.ds(off0, CH)],
                                          swa).wait()
                    pltpu.make_async_copy(g2a, g2o.at[pl.ds(off0, CH)],
                                          swa).wait()
                cpa1 = pltpu.async_copy(trow.at[idx_r.at[i0]], g1a, sga)
                cpa2 = pltpu.async_copy(tcol.at[idx_c.at[i0]], g2a, sga)

                @pl.when(g >= 1)
                def _():
                    pltpu.make_async_copy(g1b, g1o.at[pl.ds(off1, CH)],
                                          swb).wait()
                    pltpu.make_async_copy(g2b, g2o.at[pl.ds(off1, CH)],
                                          swb).wait()
                cpb1 = pltpu.async_copy(trow.at[idx_r.at[i1]], g1b, sgb)
                cpb2 = pltpu.async_copy(tcol.at[idx_c.at[i1]], g2b, sgb)

                cpa1.wait()
                cpa2.wait()
                pltpu.async_copy(g1a, g1o.at[pl.ds(off0, CH)], swa)
                pltpu.async_copy(g2a, g2o.at[pl.ds(off0, CH)], swa)
                cpb1.wait()
                cpb2.wait()
                pltpu.async_copy(g1b, g1o.at[pl.ds(off1, CH)], swb)
                pltpu.async_copy(g2b, g2o.at[pl.ds(off1, CH)], swb)
                return carry

            lax.fori_loop(0, ng, body, 0)
            pltpu.make_async_copy(g1a, g1o.at[pl.ds(base, CH)], swa).wait()
            pltpu.make_async_copy(g2a, g2o.at[pl.ds(base, CH)], swa).wait()
            pltpu.make_async_copy(g1b, g1o.at[pl.ds(base, CH)], swb).wait()
            pltpu.make_async_copy(g2b, g2o.at[pl.ds(base, CH)], swb).wait()

        return sc_gather

    sc_gather_a = make_gather(EA, EPWA, NCHA, NGA)
    sc_gather_b = make_gather(EB, EPWB, NCHB, NGB)

    @functools.partial(
        pl.kernel,
        out_type=jax.ShapeDtypeStruct((NC * NP, GD), F32),
        mesh=mesh,
        scratch_types=[
            pltpu.VMEM_SHARED((NP, GD), F32),
            pltpu.VMEM((NCH, CH), jnp.int32),
            pltpu.VMEM((CH, GD), F32),
            pltpu.SemaphoreType.DMA,
            pltpu.SemaphoreType.DMA,
        ],
    )
    def sc_count(colr, zz, onesh, outs, shared, idx2d, ones_v, saa, sab):
        c = lax.axis_index("c")
        s = lax.axis_index("s")
        w = s * NC + c
        pltpu.sync_copy(zz.at[pl.ds(s * RPT, RPT)],
                        shared.at[pl.ds(s * RPT, RPT)])
        pltpu.sync_copy(colr.at[w], idx2d)
        pltpu.sync_copy(onesh, ones_v)
        plsc.subcore_barrier()

        def body(g, carry):
            i0 = 2 * g
            i1 = 2 * g + 1

            @pl.when(g >= 1)
            def _():
                pltpu.make_async_copy(ones_v, shared.at[pl.ds(0, CH)],
                                      saa).wait()
            pltpu.async_copy(ones_v, shared.at[idx2d.at[i0]], saa, add=True)

            @pl.when(g >= 1)
            def _():
                pltpu.make_async_copy(ones_v, shared.at[pl.ds(0, CH)],
                                      sab).wait()
            pltpu.async_copy(ones_v, shared.at[idx2d.at[i1]], sab, add=True)
            return carry

        lax.fori_loop(0, NG, body, 0)
        pltpu.make_async_copy(ones_v, shared.at[pl.ds(0, CH)], saa).wait()
        pltpu.make_async_copy(ones_v, shared.at[pl.ds(0, CH)], sab).wait()
        plsc.subcore_barrier()
        pltpu.sync_copy(shared.at[pl.ds(s * RPT, RPT)],
                        outs.at[pl.ds(c * NP + s * RPT, RPT)])

    def make_scatter(epw, nch, ng):
        @functools.partial(
            pl.kernel,
            out_type=jax.ShapeDtypeStruct((NC * NP, GD), F32),
            mesh=mesh,
            scratch_types=[
                pltpu.VMEM_SHARED((NP, GD), F32),
                pltpu.VMEM((nch, CH), jnp.int32),
                pltpu.VMEM((CH, GD), F32),
                pltpu.VMEM((CH, GD), F32),
                pltpu.SemaphoreType.DMA,
                pltpu.SemaphoreType.DMA,
                pltpu.SemaphoreType.DMA,
                pltpu.SemaphoreType.DMA,
            ],
        )
        def sc_scatter(n1, colr, zz, outs, shared, idx2d,
                       vals_a, vals_b, sva, svb, saa, sab):
            c = lax.axis_index("c")
            s = lax.axis_index("s")
            w = s * NC + c
            base = w * epw
            pltpu.sync_copy(zz.at[pl.ds(s * RPT, RPT)],
                            shared.at[pl.ds(s * RPT, RPT)])
            pltpu.sync_copy(colr.at[w], idx2d)
            plsc.subcore_barrier()

            def body(g, carry):
                i0 = 2 * g
                i1 = 2 * g + 1

                @pl.when(g >= 1)
                def _():
                    pltpu.make_async_copy(vals_a, shared.at[pl.ds(0, CH)],
                                          saa).wait()
                cpa = pltpu.async_copy(n1.at[pl.ds(base + i0 * CH, CH)],
                                       vals_a, sva)

                @pl.when(g >= 1)
                def _():
                    pltpu.make_async_copy(vals_b, shared.at[pl.ds(0, CH)],
                                          sab).wait()
                cpb = pltpu.async_copy(n1.at[pl.ds(base + i1 * CH, CH)],
                                       vals_b, svb)

                cpa.wait()
                pltpu.async_copy(vals_a, shared.at[idx2d.at[i0]], saa,
                                 add=True)
                cpb.wait()
                pltpu.async_copy(vals_b, shared.at[idx2d.at[i1]], sab,
                                 add=True)
                return carry

            lax.fori_loop(0, ng, body, 0)
            pltpu.make_async_copy(vals_a, shared.at[pl.ds(0, CH)], saa).wait()
            pltpu.make_async_copy(vals_b, shared.at[pl.ds(0, CH)], sab).wait()
            plsc.subcore_barrier()
            pltpu.sync_copy(shared.at[pl.ds(s * RPT, RPT)],
                            outs.at[pl.ds(c * NP + s * RPT, RPT)])

        return sc_scatter

    sc_scatter_a = make_scatter(EPWA, NCHA, NGA)
    sc_scatter_b = make_scatter(EPWB, NCHB, NGB)

    return (sc_gather_a, sc_gather_b), sc_count, (sc_scatter_a, sc_scatter_b)

def _unused():
        return sc_gather, sc_count, sc_scatter


def _dot(a, b):
    return jnp.dot(a, b, preferred_element_type=F32)


def _pack2(a, b):
    au = jax.lax.bitcast_convert_type(a.astype(jnp.bfloat16), jnp.uint16)
    bu = jax.lax.bitcast_convert_type(b.astype(jnp.bfloat16), jnp.uint16)
    w = au.astype(jnp.uint32) | (bu.astype(jnp.uint32) << 16)
    return jax.lax.bitcast_convert_type(w, F32)


def _unpack2(p):
    w = jax.lax.bitcast_convert_type(p, jnp.uint32)
    a = jax.lax.bitcast_convert_type(w << 16, F32)
    b = jax.lax.bitcast_convert_type(w & jnp.uint32(0xFFFF0000), F32)
    return a, b


def _wspec(r, c=GD):
    return pl.BlockSpec((r, c), lambda i: (0, 0))


def _proj_body(x_ref, wr, wb, wc, trow_o, tcol_o):
    x = x_ref[...]
    trow_o[...] = _pack2(_dot(x, wr[...]), _dot(x, wb[...]))
    tcol_o[...] = _dot(x, wc[...])


_tc_proj = pl.pallas_call(
    _proj_body,
    grid=(GN,),
    in_specs=[pl.BlockSpec((BN, GD), lambda i: (i, 0)),
              _wspec(GD), _wspec(GD), _wspec(GD)],
    out_specs=[pl.BlockSpec((BN, GD), lambda i: (i, 0)),
               pl.BlockSpec((BN, GD), lambda i: (i, 0))],
    out_shape=[jax.ShapeDtypeStruct((N, GD), F32),
               jax.ShapeDtypeStruct((N, GD), F32)],
)


def _make_edge(ee, ewd, write_e):
    def body(g1, g2, ew, wew, w2e, wn1e, wn2, be1, b2e, bn1, bn2, *outs):
        ar, br = _unpack2(g1[...])
        h1 = jnp.maximum(
            ar + g2[...] + _dot(ew[...].astype(F32), wew[...]) + be1[...], 0.0)
        e = _dot(h1, w2e[...]) + b2e[...]
        h2 = jnp.maximum(br + _dot(e, wn1e[...]) + bn1[...], 0.0)
        n1v = _dot(h2, wn2[...]) + bn2[...]
        if write_e:
            outs[0][...] = e.astype(jnp.bfloat16)
            outs[1][...] = n1v
        else:
            outs[0][...] = n1v

    out_specs = [pl.BlockSpec((BE, GD), lambda i: (i, 0))]
    out_shape = [jax.ShapeDtypeStruct((ee, GD), F32)]
    if write_e:
        out_specs = out_specs * 2
        out_shape = [jax.ShapeDtypeStruct((ee, GD), jnp.bfloat16)] + out_shape
    return pl.pallas_call(
        body,
        grid=(ee // BE,),
        in_specs=[pl.BlockSpec((BE, GD), lambda i: (i, 0)),
                  pl.BlockSpec((BE, GD), lambda i: (i, 0)),
                  pl.BlockSpec((BE, ewd), lambda i: (i, 0)),
                  _wspec(ewd), _wspec(GD), _wspec(GD), _wspec(GD),
                  _wspec(1), _wspec(1), _wspec(1), _wspec(1)],
        out_specs=out_specs,
        out_shape=out_shape,
    )


_tc_edge1a = _make_edge(EA, EID, True)
_tc_edge1b = _make_edge(EB, EID, True)
_tc_edge2a = _make_edge(EA, GD, True)
_tc_edge2b = _make_edge(EB, GD, True)
_tc_edge3a = _make_edge(EA, GD, False)
_tc_edge3b = _make_edge(EB, GD, False)


def _make_node(first, last):
    def body(*refs):
        it = iter(refs)
        p0, p1, p2, p3 = next(it), next(it), next(it), next(it)
        if first:
            c0, c1 = next(it), next(it)
        else:
            inv_in = next(it)
        x_ref, wmx, wma, wm2, ubm, bm2 = (next(it) for _ in range(6))
        if not last:
            wr2, wb2, wc2 = next(it), next(it), next(it)
            xn_o, trow_o, tcol_o = next(it), next(it), next(it)
        mp_o = next(it)
        if first:
            inv_o = next(it)

        i = pl.program_id(0)
        s = (p0[0] + p1[0]) + (p2[0] + p3[0])
        if first:
            cnt = c0[0][:, :1] + c1[0][:, :1]
            invb = 1.0 / jnp.maximum(cnt, 1.0)
            inv_o[...] = jnp.broadcast_to(invb, (BN, GD))
            agg = s * invb
        else:
            agg = s * inv_in[...]
        x = x_ref[...]
        h = jnp.maximum(_dot(x, wmx[...]) + _dot(agg, wma[...]) + ubm[...], 0.0)
        xn = _dot(h, wm2[...]) + bm2[...]
        if not last:
            xn_o[...] = xn
            trow_o[...] = _pack2(_dot(xn, wr2[...]), _dot(xn, wb2[...]))
            tcol_o[...] = _dot(xn, wc2[...])

        @pl.when(i == 0)
        def _():
            mp_o[...] = jnp.zeros((8, GD), F32)

        mp_o[0:1, :] = mp_o[0:1, :] + jnp.sum(xn, axis=0, keepdims=True)

    p_spec = [pl.BlockSpec((1, BN, GD), lambda i: (0, i, 0)),
              pl.BlockSpec((1, BN, GD), lambda i: (1, i, 0)),
              pl.BlockSpec((1, BN, GD), lambda i: (0, i, 0)),
              pl.BlockSpec((1, BN, GD), lambda i: (1, i, 0))]
    in_specs = list(p_spec)
    if first:
        in_specs += [pl.BlockSpec((1, BN, GD), lambda i: (0, i, 0)),
                     pl.BlockSpec((1, BN, GD), lambda i: (1, i, 0))]
    else:
        in_specs += [pl.BlockSpec((BN, GD), lambda i: (i, 0))]
    in_specs += [pl.BlockSpec((BN, GD), lambda i: (i, 0)),
                 _wspec(GD), _wspec(GD), _wspec(GD), _wspec(1), _wspec(1)]
    if not last:
        in_specs += [_wspec(GD), _wspec(GD), _wspec(GD)]

    out_specs, out_shape = [], []
    if not last:
        out_specs += [pl.BlockSpec((BN, GD), lambda i: (i, 0)),
                      pl.BlockSpec((BN, GD), lambda i: (i, 0)),
                      pl.BlockSpec((BN, GD), lambda i: (i, 0))]
        out_shape += [jax.ShapeDtypeStruct((N, GD), F32),
                      jax.ShapeDtypeStruct((N, GD), F32),
                      jax.ShapeDtypeStruct((N, GD), F32)]
    out_specs += [pl.BlockSpec((8, GD), lambda i: (0, 0))]
    out_shape += [jax.ShapeDtypeStruct((8, GD), F32)]
    if first:
        out_specs += [pl.BlockSpec((BN, GD), lambda i: (i, 0))]
        out_shape += [jax.ShapeDtypeStruct((N, GD), F32)]
    return pl.pallas_call(body, grid=(GN,), in_specs=in_specs,
                          out_specs=out_specs, out_shape=out_shape)


_tc_node1 = _make_node(True, False)
_tc_node2 = _make_node(False, False)
_tc_node3 = _make_node(False, True)


def _make_glob(last):
    def body(mp, u, wgu, wgm, bg1, wg2, bg2, *rest):
        m = jnp.sum(mp[...], axis=0, keepdims=True) * (1.0 / N)
        m8 = jnp.broadcast_to(m, (8, GD))
        h = jnp.maximum(_dot(u[...], wgu[...]) + _dot(m8, wgm[...]) + bg1[...],
                        0.0)
        un = _dot(h, wg2[...]) + bg2[...]
        if last:
            wlin, blin, out_o = rest
            out_o[...] = (_dot(un, wlin[...]) + blin[...])[0:1, :]
        else:
            wue, b1e, wum, bm1, un_o, be1_o, ubm_o = rest
            un_o[...] = un
            be1_o[...] = (_dot(un, wue[...]) + b1e[...])[0:1, :]
            ubm_o[...] = (_dot(un, wum[...]) + bm1[...])[0:1, :]

    in_specs = [_wspec(8), _wspec(8), _wspec(GD), _wspec(GD), _wspec(1),
                _wspec(GD), _wspec(1)]
    if last:
        in_specs += [_wspec(GD), _wspec(1)]
        out_specs = [_wspec(1)]
        out_shape = [jax.ShapeDtypeStruct((1, GD), F32)]
    else:
        in_specs += [_wspec(GD), _wspec(1), _wspec(GD), _wspec(1)]
        out_specs = [_wspec(8), _wspec(1), _wspec(1)]
        out_shape = [jax.ShapeDtypeStruct((8, GD), F32),
                     jax.ShapeDtypeStruct((1, GD), F32),
                     jax.ShapeDtypeStruct((1, GD), F32)]
    return pl.pallas_call(body, grid=(1,), in_specs=in_specs,
                          out_specs=out_specs, out_shape=out_shape)


_tc_glob_mid = _make_glob(False)
_tc_glob_last = _make_glob(True)


def _split(p, ni, ewd, has_u):
    (w1e, b1e, w2e, b2e) = p['edge']
    (wn1, bn1, wn2, bn2) = p['nm1']
    (wm1, bm1, wm2, bm2) = p['nm2']
    (wg1, bg1, wg2, bg2) = p['glob']
    r = lambda b: b.reshape(1, GD)
    d = {
        'wr': w1e[:, :ni].T, 'wc': w1e[:, ni:2 * ni].T,
        'wew': w1e[:, 2 * ni:2 * ni + ewd].T,
        'b1e': r(b1e), 'w2e': w2e.T, 'b2e': r(b2e),
        'wbx': wn1[:, :ni].T, 'wn1e': wn1[:, ni:].T, 'bn1': r(bn1),
        'wn2': wn2.T, 'bn2': r(bn2),
        'wmx': wm1[:, :ni].T, 'wma': wm1[:, ni:ni + GD].T, 'bm1': r(bm1),
        'wm2': wm2.T, 'bm2': r(bm2),
        'wgu': wg1[:, :GD].T, 'wgm': wg1[:, GD:].T, 'bg1': r(bg1),
        'wg2': wg2.T, 'bg2': r(bg2),
    }
    if has_u:
        d['wue'] = w1e[:, 2 * ni + ewd:].T
        d['wum'] = wm1[:, ni + GD:].T
    return d


def kernel(x, edge_index, edge_w, params):
    (_sc_gather_a, _sc_gather_b), _sc_count, (_sc_scatter_a, _sc_scatter_b) = \
        _sc_kernels()
    row, col = edge_index[0], edge_index[1]
    q1 = _split(params['conv1'], GD, EID, False)
    q2 = _split(params['conv2'], GD, GD, True)
    q3 = _split(params['conv3'], GD, GD, True)
    wlin, blin = params['lin'][0].T, params['lin'][1].reshape(1, GD)

    u0 = jax.random.normal(jax.random.key(42), (1, GD), F32) * 0.01
    u0p = jnp.concatenate([u0, jnp.zeros((7, GD), F32)], axis=0)
    zz = jnp.zeros((NP, GD), F32)
    onesh = jnp.ones((CH, GD), F32)

    colr = col.reshape(NW, NCH, CH)
    rowra = row[:EA].reshape(NW, NCHA, CH)
    colra = col[:EA].reshape(NW, NCHA, CH)
    rowrb = row[EA:].reshape(NW, NCHB, CH)
    colrb = col[EA:].reshape(NW, NCHB, CH)
    ewa, ewb = edge_w[:EA], edge_w[EA:]

    def layer(edge_ab, scatter_extra, trow, tcol, ew_ab, qq, be1):
        edge_a, edge_b = edge_ab
        ewA, ewB = ew_ab
        g1a, g2a = _sc_gather_a(trow, tcol, rowra, colra)
        ra = edge_a(g1a, g2a, ewA, qq['wew'], qq['w2e'], qq['wn1e'],
                    qq['wn2'], be1, qq['b2e'], qq['bn1'], qq['bn2'])
        g1b, g2b = _sc_gather_b(trow, tcol, rowrb, colrb)
        rb = edge_b(g1b, g2b, ewB, qq['wew'], qq['w2e'], qq['wn1e'],
                    qq['wn2'], be1, qq['b2e'], qq['bn1'], qq['bn2'])
        psa = _sc_scatter_a(ra[-1], colra, zz)
        psb = _sc_scatter_b(rb[-1], colrb, zz)
        e_ab = (ra[0], rb[0]) if len(ra) == 2 else None
        return e_ab, psa, psb

    trow, tcol = _tc_proj(x, q1['wr'], q1['wbx'], q1['wc'])
    pc = _sc_count(colr, zz, onesh)
    e1, psa, psb = layer((_tc_edge1a, _tc_edge1b), None, trow, tcol,
                         (ewa, ewb), q1, q1['b1e'])
    xn1, trow, tcol, mp, inv = _tc_node1(
        psa.reshape(2, NP, GD), psa.reshape(2, NP, GD),
        psb.reshape(2, NP, GD), psb.reshape(2, NP, GD),
        pc.reshape(2, NP, GD), pc.reshape(2, NP, GD),
        x, q1['wmx'], q1['wma'], q1['wm2'], q1['bm1'], q1['bm2'],
        q2['wr'], q2['wbx'], q2['wc'])
    u1, be1_2, ubm2 = _tc_glob_mid(mp, u0p, q1['wgu'], q1['wgm'], q1['bg1'],
                                   q1['wg2'], q1['bg2'],
                                   q2['wue'], q2['b1e'], q2['wum'], q2['bm1'])

    e2, psa, psb = layer((_tc_edge2a, _tc_edge2b), None, trow, tcol,
                         e1, q2, be1_2)
    xn2, trow, tcol, mp = _tc_node2(
        psa.reshape(2, NP, GD), psa.reshape(2, NP, GD),
        psb.reshape(2, NP, GD), psb.reshape(2, NP, GD), inv,
        xn1, q2['wmx'], q2['wma'], q2['wm2'], ubm2, q2['bm2'],
        q3['wr'], q3['wbx'], q3['wc'])
    u2, be1_3, ubm3 = _tc_glob_mid(mp, u1, q2['wgu'], q2['wgm'], q2['bg1'],
                                   q2['wg2'], q2['bg2'],
                                   q3['wue'], q3['b1e'], q3['wum'], q3['bm1'])

    _, psa, psb = layer((_tc_edge3a, _tc_edge3b), None, trow, tcol,
                        e2, q3, be1_3)
    (mp,) = _tc_node3(
        psa.reshape(2, NP, GD), psa.reshape(2, NP, GD),
        psb.reshape(2, NP, GD), psb.reshape(2, NP, GD), inv,
        xn2, q3['wmx'], q3['wma'], q3['wm2'], ubm3, q3['bm2'])
    (out,) = _tc_glob_last(mp, u2, q3['wgu'], q3['wgm'], q3['bg1'],
                           q3['wg2'], q3['bg2'], wlin, blin)
    return out


# speedup vs baseline: 2.9144x; 1.0729x over previous
import functools
import jax
import jax.numpy as jnp
from jax import lax
from jax.experimental import pallas as pl
from jax.experimental.pallas import tpu as pltpu
from jax.experimental.pallas import tpu_sc as plsc

N = 10000
E = 320000
GD = 128
EID = 16
F32 = jnp.float32

NC, NS = 2, 16
NW = NC * NS
EPW = E // NW
CH = 40
NCH = EPW // CH
NG = NCH // 2
PARTS = (81920, 81920, 81920, 74240)
POFF = (0, 81920, 163840, 245760)
NP = 10240
RPT = NP // NS

BE = 512
GE = E // BE
BN = 2000
GN = N // BN

@functools.lru_cache(maxsize=None)
def _sc_kernels():
    mesh = plsc.VectorSubcoreMesh(core_axis_name="c", subcore_axis_name="s",
                                  num_cores=NC, num_subcores=NS)

    def make_gather(ee, epw, nch, ng):
        @functools.partial(
            pl.kernel,
            out_type=[jax.ShapeDtypeStruct((ee, GD), F32),
                      jax.ShapeDtypeStruct((ee, GD), F32)],
            mesh=mesh,
            scratch_types=[
                pltpu.VMEM((nch, CH), jnp.int32),
                pltpu.VMEM((nch, CH), jnp.int32),
                pltpu.VMEM((CH, GD), F32),
                pltpu.VMEM((CH, GD), F32),
                pltpu.VMEM((CH, GD), F32),
                pltpu.VMEM((CH, GD), F32),
                pltpu.SemaphoreType.DMA,
                pltpu.SemaphoreType.DMA,
                pltpu.SemaphoreType.DMA,
                pltpu.SemaphoreType.DMA,
            ],
        )
        def sc_gather(trow, tcol, rowr, colr, g1o, g2o,
                      idx_r, idx_c, g1a, g1b, g2a, g2b, sga, sgb, swa, swb):
            c = lax.axis_index("c")
            s = lax.axis_index("s")
            w = s * NC + c
            base = w * epw
            pltpu.sync_copy(rowr.at[w], idx_r)
            pltpu.sync_copy(colr.at[w], idx_c)

            def body(g, carry):
                i0 = 2 * g
                i1 = 2 * g + 1
                off0 = base + i0 * CH
                off1 = base + i1 * CH

                @pl.when(g >= 1)
                def _():
                    pltpu.make_async_copy(g1a, g1o.at[pl.ds(off0, CH)],
                                          swa).wait()
                    pltpu.make_async_copy(g2a, g2o.at[pl.ds(off0, CH)],
                                          swa).wait()
                cpa1 = pltpu.async_copy(trow.at[idx_r.at[i0]], g1a, sga)
                cpa2 = pltpu.async_copy(tcol.at[idx_c.at[i0]], g2a, sga)

                @pl.when(g >= 1)
                def _():
                    pltpu.make_async_copy(g1b, g1o.at[pl.ds(off1, CH)],
                                          swb).wait()
                    pltpu.make_async_copy(g2b, g2o.at[pl.ds(off1, CH)],
                                          swb).wait()
                cpb1 = pltpu.async_copy(trow.at[idx_r.at[i1]], g1b, sgb)
                cpb2 = pltpu.async_copy(tcol.at[idx_c.at[i1]], g2b, sgb)

                cpa1.wait()
                cpa2.wait()
                pltpu.async_copy(g1a, g1o.at[pl.ds(off0, CH)], swa)
                pltpu.async_copy(g2a, g2o.at[pl.ds(off0, CH)], swa)
                cpb1.wait()
                cpb2.wait()
                pltpu.async_copy(g1b, g1o.at[pl.ds(off1, CH)], swb)
                pltpu.async_copy(g2b, g2o.at[pl.ds(off1, CH)], swb)
                return carry

            lax.fori_loop(0, ng, body, 0)
            pltpu.make_async_copy(g1a, g1o.at[pl.ds(base, CH)], swa).wait()
            pltpu.make_async_copy(g2a, g2o.at[pl.ds(base, CH)], swa).wait()
            pltpu.make_async_copy(g1b, g1o.at[pl.ds(base, CH)], swb).wait()
            pltpu.make_async_copy(g2b, g2o.at[pl.ds(base, CH)], swb).wait()

        return sc_gather

    gathers = tuple(
        make_gather(pe, pe // NW, pe // NW // CH, pe // NW // CH // 2)
        for pe in PARTS)

    @functools.partial(
        pl.kernel,
        out_type=jax.ShapeDtypeStruct((NC * NP, GD), F32),
        mesh=mesh,
        scratch_types=[
            pltpu.VMEM_SHARED((NP, GD), F32),
            pltpu.VMEM((NCH, CH), jnp.int32),
            pltpu.VMEM((CH, GD), F32),
            pltpu.SemaphoreType.DMA,
            pltpu.SemaphoreType.DMA,
        ],
    )
    def sc_count(colr, zz, onesh, outs, shared, idx2d, ones_v, saa, sab):
        c = lax.axis_index("c")
        s = lax.axis_index("s")
        w = s * NC + c
        pltpu.sync_copy(zz.at[pl.ds(s * RPT, RPT)],
                        shared.at[pl.ds(s * RPT, RPT)])
        pltpu.sync_copy(colr.at[w], idx2d)
        pltpu.sync_copy(onesh, ones_v)
        plsc.subcore_barrier()

        def body(g, carry):
            i0 = 2 * g
            i1 = 2 * g + 1

            @pl.when(g >= 1)
            def _():
                pltpu.make_async_copy(ones_v, shared.at[pl.ds(0, CH)],
                                      saa).wait()
            pltpu.async_copy(ones_v, shared.at[idx2d.at[i0]], saa, add=True)

            @pl.when(g >= 1)
            def _():
                pltpu.make_async_copy(ones_v, shared.at[pl.ds(0, CH)],
                                      sab).wait()
            pltpu.async_copy(ones_v, shared.at[idx2d.at[i1]], sab, add=True)
            return carry

        lax.fori_loop(0, NG, body, 0)
        pltpu.make_async_copy(ones_v, shared.at[pl.ds(0, CH)], saa).wait()
        pltpu.make_async_copy(ones_v, shared.at[pl.ds(0, CH)], sab).wait()
        plsc.subcore_barrier()
        pltpu.sync_copy(shared.at[pl.ds(s * RPT, RPT)],
                        outs.at[pl.ds(c * NP + s * RPT, RPT)])

    def make_scatter(epw, nch, ng):
        @functools.partial(
            pl.kernel,
            out_type=jax.ShapeDtypeStruct((NC * NP, GD), F32),
            mesh=mesh,
            scratch_types=[
                pltpu.VMEM_SHARED((NP, GD), F32),
                pltpu.VMEM((nch, CH), jnp.int32),
                pltpu.VMEM((CH, GD), F32),
                pltpu.VMEM((CH, GD), F32),
                pltpu.SemaphoreType.DMA,
                pltpu.SemaphoreType.DMA,
                pltpu.SemaphoreType.DMA,
                pltpu.SemaphoreType.DMA,
            ],
        )
        def sc_scatter(n1, colr, zz, outs, shared, idx2d,
                       vals_a, vals_b, sva, svb, saa, sab):
            c = lax.axis_index("c")
            s = lax.axis_index("s")
            w = s * NC + c
            base = w * epw
            pltpu.sync_copy(zz.at[pl.ds(s * RPT, RPT)],
                            shared.at[pl.ds(s * RPT, RPT)])
            pltpu.sync_copy(colr.at[w], idx2d)
            plsc.subcore_barrier()

            def body(g, carry):
                i0 = 2 * g
                i1 = 2 * g + 1

                @pl.when(g >= 1)
                def _():
                    pltpu.make_async_copy(vals_a, shared.at[pl.ds(0, CH)],
                                          saa).wait()
                cpa = pltpu.async_copy(n1.at[pl.ds(base + i0 * CH, CH)],
                                       vals_a, sva)

                @pl.when(g >= 1)
                def _():
                    pltpu.make_async_copy(vals_b, shared.at[pl.ds(0, CH)],
                                          sab).wait()
                cpb = pltpu.async_copy(n1.at[pl.ds(base + i1 * CH, CH)],
                                       vals_b, svb)

                cpa.wait()
                pltpu.async_copy(vals_a, shared.at[idx2d.at[i0]], saa,
                                 add=True)
                cpb.wait()
                pltpu.async_copy(vals_b, shared.at[idx2d.at[i1]], sab,
                                 add=True)
                return carry

            lax.fori_loop(0, ng, body, 0)
            pltpu.make_async_copy(vals_a, shared.at[pl.ds(0, CH)], saa).wait()
            pltpu.make_async_copy(vals_b, shared.at[pl.ds(0, CH)], sab).wait()
            plsc.subcore_barrier()
            pltpu.sync_copy(shared.at[pl.ds(s * RPT, RPT)],
                            outs.at[pl.ds(c * NP + s * RPT, RPT)])

        return sc_scatter

    scatters = tuple(
        make_scatter(pe // NW, pe // NW // CH, pe // NW // CH // 2)
        for pe in PARTS)

    return gathers, sc_count, scatters

def _unused():
        return sc_gather, sc_count, sc_scatter


def _dot(a, b):
    return jnp.dot(a, b, preferred_element_type=F32)


def _pack2(a, b):
    au = jax.lax.bitcast_convert_type(a.astype(jnp.bfloat16), jnp.uint16)
    bu = jax.lax.bitcast_convert_type(b.astype(jnp.bfloat16), jnp.uint16)
    w = au.astype(jnp.uint32) | (bu.astype(jnp.uint32) << 16)
    return jax.lax.bitcast_convert_type(w, F32)


def _unpack2(p):
    w = jax.lax.bitcast_convert_type(p, jnp.uint32)
    a = jax.lax.bitcast_convert_type(w << 16, F32)
    b = jax.lax.bitcast_convert_type(w & jnp.uint32(0xFFFF0000), F32)
    return a, b


def _wspec(r, c=GD):
    return pl.BlockSpec((r, c), lambda i: (0, 0))


def _proj_body(x_ref, wr, wb, wc, trow_o, tcol_o):
    x = x_ref[...]
    trow_o[...] = _pack2(_dot(x, wr[...]), _dot(x, wb[...]))
    tcol_o[...] = _dot(x, wc[...])


_tc_proj = pl.pallas_call(
    _proj_body,
    grid=(GN,),
    in_specs=[pl.BlockSpec((BN, GD), lambda i: (i, 0)),
              _wspec(GD), _wspec(GD), _wspec(GD)],
    out_specs=[pl.BlockSpec((BN, GD), lambda i: (i, 0)),
               pl.BlockSpec((BN, GD), lambda i: (i, 0))],
    out_shape=[jax.ShapeDtypeStruct((N, GD), F32),
               jax.ShapeDtypeStruct((N, GD), F32)],
)


def _make_edge(ee, ewd, write_e):
    def body(g1, g2, ew, wew, w2e, wn1e, wn2, be1, b2e, bn1, bn2, *outs):
        ar, br = _unpack2(g1[...])
        h1 = jnp.maximum(
            ar + g2[...] + _dot(ew[...].astype(F32), wew[...]) + be1[...], 0.0)
        e = _dot(h1, w2e[...]) + b2e[...]
        h2 = jnp.maximum(br + _dot(e, wn1e[...]) + bn1[...], 0.0)
        n1v = _dot(h2, wn2[...]) + bn2[...]
        if write_e:
            outs[0][...] = e.astype(jnp.bfloat16)
            outs[1][...] = n1v
        else:
            outs[0][...] = n1v

    out_specs = [pl.BlockSpec((BE, GD), lambda i: (i, 0))]
    out_shape = [jax.ShapeDtypeStruct((ee, GD), F32)]
    if write_e:
        out_specs = out_specs * 2
        out_shape = [jax.ShapeDtypeStruct((ee, GD), jnp.bfloat16)] + out_shape
    return pl.pallas_call(
        body,
        grid=(ee // BE,),
        in_specs=[pl.BlockSpec((BE, GD), lambda i: (i, 0)),
                  pl.BlockSpec((BE, GD), lambda i: (i, 0)),
                  pl.BlockSpec((BE, ewd), lambda i: (i, 0)),
                  _wspec(ewd), _wspec(GD), _wspec(GD), _wspec(GD),
                  _wspec(1), _wspec(1), _wspec(1), _wspec(1)],
        out_specs=out_specs,
        out_shape=out_shape,
    )


_tc_edges1 = tuple(_make_edge(pe, EID, True) for pe in PARTS)
_tc_edges2 = tuple(_make_edge(pe, GD, True) for pe in PARTS)
_tc_edges3 = tuple(_make_edge(pe, GD, False) for pe in PARTS)


def _make_node(first, last):
    def body(*refs):
        it = iter(refs)
        ps = [next(it) for _ in range(8)]
        if first:
            c0, c1 = next(it), next(it)
        else:
            inv_in = next(it)
        x_ref, wmx, wma, wm2, ubm, bm2 = (next(it) for _ in range(6))
        if not last:
            wr2, wb2, wc2 = next(it), next(it), next(it)
            xn_o, trow_o, tcol_o = next(it), next(it), next(it)
        mp_o = next(it)
        if first:
            inv_o = next(it)

        i = pl.program_id(0)
        s = ((ps[0][0] + ps[1][0]) + (ps[2][0] + ps[3][0]) +
             (ps[4][0] + ps[5][0]) + (ps[6][0] + ps[7][0]))
        if first:
            cnt = c0[0][:, :1] + c1[0][:, :1]
            invb = 1.0 / jnp.maximum(cnt, 1.0)
            inv_o[...] = jnp.broadcast_to(invb, (BN, GD))
            agg = s * invb
        else:
            agg = s * inv_in[...]
        x = x_ref[...]
        h = jnp.maximum(_dot(x, wmx[...]) + _dot(agg, wma[...]) + ubm[...], 0.0)
        xn = _dot(h, wm2[...]) + bm2[...]
        if not last:
            xn_o[...] = xn
            trow_o[...] = _pack2(_dot(xn, wr2[...]), _dot(xn, wb2[...]))
            tcol_o[...] = _dot(xn, wc2[...])

        @pl.when(i == 0)
        def _():
            mp_o[...] = jnp.zeros((8, GD), F32)

        mp_o[0:1, :] = mp_o[0:1, :] + jnp.sum(xn, axis=0, keepdims=True)

    p_spec = [pl.BlockSpec((1, BN, GD), lambda i: (0, i, 0)),
              pl.BlockSpec((1, BN, GD), lambda i: (1, i, 0))] * 4
    in_specs = list(p_spec)
    if first:
        in_specs += [pl.BlockSpec((1, BN, GD), lambda i: (0, i, 0)),
                     pl.BlockSpec((1, BN, GD), lambda i: (1, i, 0))]
    else:
        in_specs += [pl.BlockSpec((BN, GD), lambda i: (i, 0))]
    in_specs += [pl.BlockSpec((BN, GD), lambda i: (i, 0)),
                 _wspec(GD), _wspec(GD), _wspec(GD), _wspec(1), _wspec(1)]
    if not last:
        in_specs += [_wspec(GD), _wspec(GD), _wspec(GD)]

    out_specs, out_shape = [], []
    if not last:
        out_specs += [pl.BlockSpec((BN, GD), lambda i: (i, 0)),
                      pl.BlockSpec((BN, GD), lambda i: (i, 0)),
                      pl.BlockSpec((BN, GD), lambda i: (i, 0))]
        out_shape += [jax.ShapeDtypeStruct((N, GD), F32),
                      jax.ShapeDtypeStruct((N, GD), F32),
                      jax.ShapeDtypeStruct((N, GD), F32)]
    out_specs += [pl.BlockSpec((8, GD), lambda i: (0, 0))]
    out_shape += [jax.ShapeDtypeStruct((8, GD), F32)]
    if first:
        out_specs += [pl.BlockSpec((BN, GD), lambda i: (i, 0))]
        out_shape += [jax.ShapeDtypeStruct((N, GD), F32)]
    return pl.pallas_call(body, grid=(GN,), in_specs=in_specs,
                          out_specs=out_specs, out_shape=out_shape)


_tc_node1 = _make_node(True, False)
_tc_node2 = _make_node(False, False)
_tc_node3 = _make_node(False, True)


def _make_glob(last):
    def body(mp, u, wgu, wgm, bg1, wg2, bg2, *rest):
        m = jnp.sum(mp[...], axis=0, keepdims=True) * (1.0 / N)
        m8 = jnp.broadcast_to(m, (8, GD))
        h = jnp.maximum(_dot(u[...], wgu[...]) + _dot(m8, wgm[...]) + bg1[...],
                        0.0)
        un = _dot(h, wg2[...]) + bg2[...]
        if last:
            wlin, blin, out_o = rest
            out_o[...] = (_dot(un, wlin[...]) + blin[...])[0:1, :]
        else:
            wue, b1e, wum, bm1, un_o, be1_o, ubm_o = rest
            un_o[...] = un
            be1_o[...] = (_dot(un, wue[...]) + b1e[...])[0:1, :]
            ubm_o[...] = (_dot(un, wum[...]) + bm1[...])[0:1, :]

    in_specs = [_wspec(8), _wspec(8), _wspec(GD), _wspec(GD), _wspec(1),
                _wspec(GD), _wspec(1)]
    if last:
        in_specs += [_wspec(GD), _wspec(1)]
        out_specs = [_wspec(1)]
        out_shape = [jax.ShapeDtypeStruct((1, GD), F32)]
    else:
        in_specs += [_wspec(GD), _wspec(1), _wspec(GD), _wspec(1)]
        out_specs = [_wspec(8), _wspec(1), _wspec(1)]
        out_shape = [jax.ShapeDtypeStruct((8, GD), F32),
                     jax.ShapeDtypeStruct((1, GD), F32),
                     jax.ShapeDtypeStruct((1, GD), F32)]
    return pl.pallas_call(body, grid=(1,), in_specs=in_specs,
                          out_specs=out_specs, out_shape=out_shape)


_tc_glob_mid = _make_glob(False)
_tc_glob_last = _make_glob(True)


def _split(p, ni, ewd, has_u):
    (w1e, b1e, w2e, b2e) = p['edge']
    (wn1, bn1, wn2, bn2) = p['nm1']
    (wm1, bm1, wm2, bm2) = p['nm2']
    (wg1, bg1, wg2, bg2) = p['glob']
    r = lambda b: b.reshape(1, GD)
    d = {
        'wr': w1e[:, :ni].T, 'wc': w1e[:, ni:2 * ni].T,
        'wew': w1e[:, 2 * ni:2 * ni + ewd].T,
        'b1e': r(b1e), 'w2e': w2e.T, 'b2e': r(b2e),
        'wbx': wn1[:, :ni].T, 'wn1e': wn1[:, ni:].T, 'bn1': r(bn1),
        'wn2': wn2.T, 'bn2': r(bn2),
        'wmx': wm1[:, :ni].T, 'wma': wm1[:, ni:ni + GD].T, 'bm1': r(bm1),
        'wm2': wm2.T, 'bm2': r(bm2),
        'wgu': wg1[:, :GD].T, 'wgm': wg1[:, GD:].T, 'bg1': r(bg1),
        'wg2': wg2.T, 'bg2': r(bg2),
    }
    if has_u:
        d['wue'] = w1e[:, 2 * ni + ewd:].T
        d['wum'] = wm1[:, ni + GD:].T
    return d


def kernel(x, edge_index, edge_w, params):
    _sc_gathers, _sc_count, _sc_scatters = _sc_kernels()
    row, col = edge_index[0], edge_index[1]
    q1 = _split(params['conv1'], GD, EID, False)
    q2 = _split(params['conv2'], GD, GD, True)
    q3 = _split(params['conv3'], GD, GD, True)
    wlin, blin = params['lin'][0].T, params['lin'][1].reshape(1, GD)

    u0 = jax.random.normal(jax.random.key(42), (1, GD), F32) * 0.01
    u0p = jnp.concatenate([u0, jnp.zeros((7, GD), F32)], axis=0)
    zz = jnp.zeros((NP, GD), F32)
    onesh = jnp.ones((CH, GD), F32)

    colr = col.reshape(NW, NCH, CH)
    rowrs, colrs, ews = [], [], []
    for pe, po in zip(PARTS, POFF):
        rowrs.append(lax.dynamic_slice(row, (po,), (pe,)).reshape(
            NW, pe // NW // CH, CH))
        colrs.append(lax.dynamic_slice(col, (po,), (pe,)).reshape(
            NW, pe // NW // CH, CH))
        ews.append(lax.dynamic_slice(edge_w, (po, 0), (pe, EID)))

    def layer(edges, trow, tcol, ew_parts, qq, be1):
        rs, pss = [], []
        for p in range(len(PARTS)):
            g1, g2 = _sc_gathers[p](trow, tcol, rowrs[p], colrs[p])
            rs.append(edges[p](g1, g2, ew_parts[p], qq['wew'], qq['w2e'],
                               qq['wn1e'], qq['wn2'], be1, qq['b2e'],
                               qq['bn1'], qq['bn2']))
        for p in range(len(PARTS)):
            pss.append(_sc_scatters[p](rs[p][-1], colrs[p], zz))
        e_parts = tuple(r[0] for r in rs) if len(rs[0]) == 2 else None
        return e_parts, pss

    def pargs(pss):
        out = []
        for ps in pss:
            r = ps.reshape(2, NP, GD)
            out += [r, r]
        return out

    trow, tcol = _tc_proj(x, q1['wr'], q1['wbx'], q1['wc'])
    pc = _sc_count(colr, zz, onesh)
    e1, pss = layer(_tc_edges1, trow, tcol, ews, q1, q1['b1e'])
    xn1, trow, tcol, mp, inv = _tc_node1(
        *pargs(pss), pc.reshape(2, NP, GD), pc.reshape(2, NP, GD),
        x, q1['wmx'], q1['wma'], q1['wm2'], q1['bm1'], q1['bm2'],
        q2['wr'], q2['wbx'], q2['wc'])
    u1, be1_2, ubm2 = _tc_glob_mid(mp, u0p, q1['wgu'], q1['wgm'], q1['bg1'],
                                   q1['wg2'], q1['bg2'],
                                   q2['wue'], q2['b1e'], q2['wum'], q2['bm1'])

    e2, pss = layer(_tc_edges2, trow, tcol, e1, q2, be1_2)
    xn2, trow, tcol, mp = _tc_node2(
        *pargs(pss), inv,
        xn1, q2['wmx'], q2['wma'], q2['wm2'], ubm2, q2['bm2'],
        q3['wr'], q3['wbx'], q3['wc'])
    u2, be1_3, ubm3 = _tc_glob_mid(mp, u1, q2['wgu'], q2['wgm'], q2['bg1'],
                                   q2['wg2'], q2['bg2'],
                                   q3['wue'], q3['b1e'], q3['wum'], q3['bm1'])

    _, pss = layer(_tc_edges3, trow, tcol, e2, q3, be1_3)
    (mp,) = _tc_node3(
        *pargs(pss), inv,
        xn2, q3['wmx'], q3['wma'], q3['wm2'], ubm3, q3['bm2'])
    (out,) = _tc_glob_last(mp, u2, q3['wgu'], q3['wgm'], q3['bg1'],
                           q3['wg2'], q3['bg2'], wlin, blin)
    return out


# speedup vs baseline: 2.9184x; 1.0014x over previous
import functools
import jax
import jax.numpy as jnp
from jax import lax
from jax.experimental import pallas as pl
from jax.experimental.pallas import tpu as pltpu
from jax.experimental.pallas import tpu_sc as plsc

N = 10000
E = 320000
GD = 128
EID = 16
F32 = jnp.float32

NC, NS = 2, 16
NW = NC * NS
EPW = E // NW
CH = 40
NCH = EPW // CH
NG = NCH // 2
PARTS = (40960, 40960, 40960, 40960, 40960, 40960, 40960, 33280)
POFF = (0, 40960, 81920, 122880, 163840, 204800, 245760, 286720)
NP = 10240
RPT = NP // NS

BE = 512
GE = E // BE
BN = 2000
GN = N // BN

@functools.lru_cache(maxsize=None)
def _sc_kernels():
    mesh = plsc.VectorSubcoreMesh(core_axis_name="c", subcore_axis_name="s",
                                  num_cores=NC, num_subcores=NS)

    def make_gather(ee, epw, nch, ng):
        @functools.partial(
            pl.kernel,
            out_type=[jax.ShapeDtypeStruct((ee, GD), F32),
                      jax.ShapeDtypeStruct((ee, GD), F32)],
            mesh=mesh,
            scratch_types=[
                pltpu.VMEM((nch, CH), jnp.int32),
                pltpu.VMEM((nch, CH), jnp.int32),
                pltpu.VMEM((CH, GD), F32),
                pltpu.VMEM((CH, GD), F32),
                pltpu.VMEM((CH, GD), F32),
                pltpu.VMEM((CH, GD), F32),
                pltpu.SemaphoreType.DMA,
                pltpu.SemaphoreType.DMA,
                pltpu.SemaphoreType.DMA,
                pltpu.SemaphoreType.DMA,
            ],
        )
        def sc_gather(trow, tcol, rowr, colr, g1o, g2o,
                      idx_r, idx_c, g1a, g1b, g2a, g2b, sga, sgb, swa, swb):
            c = lax.axis_index("c")
            s = lax.axis_index("s")
            w = s * NC + c
            base = w * epw
            pltpu.sync_copy(rowr.at[w], idx_r)
            pltpu.sync_copy(colr.at[w], idx_c)

            def body(g, carry):
                i0 = 2 * g
                i1 = 2 * g + 1
                off0 = base + i0 * CH
                off1 = base + i1 * CH

                @pl.when(g >= 1)
                def _():
                    pltpu.make_async_copy(g1a, g1o.at[pl.ds(off0, CH)],
                                          swa).wait()
                    pltpu.make_async_copy(g2a, g2o.at[pl.ds(off0, CH)],
                                          swa).wait()
                cpa1 = pltpu.async_copy(trow.at[idx_r.at[i0]], g1a, sga)
                cpa2 = pltpu.async_copy(tcol.at[idx_c.at[i0]], g2a, sga)

                @pl.when(g >= 1)
                def _():
                    pltpu.make_async_copy(g1b, g1o.at[pl.ds(off1, CH)],
                                          swb).wait()
                    pltpu.make_async_copy(g2b, g2o.at[pl.ds(off1, CH)],
                                          swb).wait()
                cpb1 = pltpu.async_copy(trow.at[idx_r.at[i1]], g1b, sgb)
                cpb2 = pltpu.async_copy(tcol.at[idx_c.at[i1]], g2b, sgb)

                cpa1.wait()
                cpa2.wait()
                pltpu.async_copy(g1a, g1o.at[pl.ds(off0, CH)], swa)
                pltpu.async_copy(g2a, g2o.at[pl.ds(off0, CH)], swa)
                cpb1.wait()
                cpb2.wait()
                pltpu.async_copy(g1b, g1o.at[pl.ds(off1, CH)], swb)
                pltpu.async_copy(g2b, g2o.at[pl.ds(off1, CH)], swb)
                return carry

            lax.fori_loop(0, ng, body, 0)
            pltpu.make_async_copy(g1a, g1o.at[pl.ds(base, CH)], swa).wait()
            pltpu.make_async_copy(g2a, g2o.at[pl.ds(base, CH)], swa).wait()
            pltpu.make_async_copy(g1b, g1o.at[pl.ds(base, CH)], swb).wait()
            pltpu.make_async_copy(g2b, g2o.at[pl.ds(base, CH)], swb).wait()

        return sc_gather

    gathers = tuple(
        make_gather(pe, pe // NW, pe // NW // CH, pe // NW // CH // 2)
        for pe in PARTS)

    @functools.partial(
        pl.kernel,
        out_type=jax.ShapeDtypeStruct((NC * NP, GD), F32),
        mesh=mesh,
        scratch_types=[
            pltpu.VMEM_SHARED((NP, GD), F32),
            pltpu.VMEM((NCH, CH), jnp.int32),
            pltpu.VMEM((CH, GD), F32),
            pltpu.SemaphoreType.DMA,
            pltpu.SemaphoreType.DMA,
        ],
    )
    def sc_count(colr, zz, onesh, outs, shared, idx2d, ones_v, saa, sab):
        c = lax.axis_index("c")
        s = lax.axis_index("s")
        w = s * NC + c
        pltpu.sync_copy(zz.at[pl.ds(s * RPT, RPT)],
                        shared.at[pl.ds(s * RPT, RPT)])
        pltpu.sync_copy(colr.at[w], idx2d)
        pltpu.sync_copy(onesh, ones_v)
        plsc.subcore_barrier()

        def body(g, carry):
            i0 = 2 * g
            i1 = 2 * g + 1

            @pl.when(g >= 1)
            def _():
                pltpu.make_async_copy(ones_v, shared.at[pl.ds(0, CH)],
                                      saa).wait()
            pltpu.async_copy(ones_v, shared.at[idx2d.at[i0]], saa, add=True)

            @pl.when(g >= 1)
            def _():
                pltpu.make_async_copy(ones_v, shared.at[pl.ds(0, CH)],
                                      sab).wait()
            pltpu.async_copy(ones_v, shared.at[idx2d.at[i1]], sab, add=True)
            return carry

        lax.fori_loop(0, NG, body, 0)
        pltpu.make_async_copy(ones_v, shared.at[pl.ds(0, CH)], saa).wait()
        pltpu.make_async_copy(ones_v, shared.at[pl.ds(0, CH)], sab).wait()
        plsc.subcore_barrier()
        pltpu.sync_copy(shared.at[pl.ds(s * RPT, RPT)],
                        outs.at[pl.ds(c * NP + s * RPT, RPT)])

    def make_scatter(epw, nch, ng):
        @functools.partial(
            pl.kernel,
            out_type=jax.ShapeDtypeStruct((NC * NP, GD), F32),
            mesh=mesh,
            scratch_types=[
                pltpu.VMEM_SHARED((NP, GD), F32),
                pltpu.VMEM((nch, CH), jnp.int32),
                pltpu.VMEM((CH, GD), F32),
                pltpu.VMEM((CH, GD), F32),
                pltpu.SemaphoreType.DMA,
                pltpu.SemaphoreType.DMA,
                pltpu.SemaphoreType.DMA,
                pltpu.SemaphoreType.DMA,
            ],
        )
        def sc_scatter(n1, colr, zz, outs, shared, idx2d,
                       vals_a, vals_b, sva, svb, saa, sab):
            c = lax.axis_index("c")
            s = lax.axis_index("s")
            w = s * NC + c
            base = w * epw
            pltpu.sync_copy(zz.at[pl.ds(s * RPT, RPT)],
                            shared.at[pl.ds(s * RPT, RPT)])
            pltpu.sync_copy(colr.at[w], idx2d)
            plsc.subcore_barrier()

            def body(g, carry):
                i0 = 2 * g
                i1 = 2 * g + 1

                @pl.when(g >= 1)
                def _():
                    pltpu.make_async_copy(vals_a, shared.at[pl.ds(0, CH)],
                                          saa).wait()
                cpa = pltpu.async_copy(n1.at[pl.ds(base + i0 * CH, CH)],
                                       vals_a, sva)

                @pl.when(g >= 1)
                def _():
                    pltpu.make_async_copy(vals_b, shared.at[pl.ds(0, CH)],
                                          sab).wait()
                cpb = pltpu.async_copy(n1.at[pl.ds(base + i1 * CH, CH)],
                                       vals_b, svb)

                cpa.wait()
                pltpu.async_copy(vals_a, shared.at[idx2d.at[i0]], saa,
                                 add=True)
                cpb.wait()
                pltpu.async_copy(vals_b, shared.at[idx2d.at[i1]], sab,
                                 add=True)
                return carry

            lax.fori_loop(0, ng, body, 0)
            pltpu.make_async_copy(vals_a, shared.at[pl.ds(0, CH)], saa).wait()
            pltpu.make_async_copy(vals_b, shared.at[pl.ds(0, CH)], sab).wait()
            plsc.subcore_barrier()
            pltpu.sync_copy(shared.at[pl.ds(s * RPT, RPT)],
                            outs.at[pl.ds(c * NP + s * RPT, RPT)])

        return sc_scatter

    scatters = tuple(
        make_scatter(pe // NW, pe // NW // CH, pe // NW // CH // 2)
        for pe in PARTS)

    return gathers, sc_count, scatters

def _unused():
        return sc_gather, sc_count, sc_scatter


def _dot(a, b):
    return jnp.dot(a, b, preferred_element_type=F32)


def _pack2(a, b):
    au = jax.lax.bitcast_convert_type(a.astype(jnp.bfloat16), jnp.uint16)
    bu = jax.lax.bitcast_convert_type(b.astype(jnp.bfloat16), jnp.uint16)
    w = au.astype(jnp.uint32) | (bu.astype(jnp.uint32) << 16)
    return jax.lax.bitcast_convert_type(w, F32)


def _unpack2(p):
    w = jax.lax.bitcast_convert_type(p, jnp.uint32)
    a = jax.lax.bitcast_convert_type(w << 16, F32)
    b = jax.lax.bitcast_convert_type(w & jnp.uint32(0xFFFF0000), F32)
    return a, b


def _wspec(r, c=GD):
    return pl.BlockSpec((r, c), lambda i: (0, 0))


def _proj_body(x_ref, wr, wb, wc, trow_o, tcol_o):
    x = x_ref[...]
    trow_o[...] = _pack2(_dot(x, wr[...]), _dot(x, wb[...]))
    tcol_o[...] = _dot(x, wc[...])


_tc_proj = pl.pallas_call(
    _proj_body,
    grid=(GN,),
    in_specs=[pl.BlockSpec((BN, GD), lambda i: (i, 0)),
              _wspec(GD), _wspec(GD), _wspec(GD)],
    out_specs=[pl.BlockSpec((BN, GD), lambda i: (i, 0)),
               pl.BlockSpec((BN, GD), lambda i: (i, 0))],
    out_shape=[jax.ShapeDtypeStruct((N, GD), F32),
               jax.ShapeDtypeStruct((N, GD), F32)],
)


def _make_edge(ee, ewd, write_e):
    def body(g1, g2, ew, wew, w2e, wn1e, wn2, be1, b2e, bn1, bn2, *outs):
        ar, br = _unpack2(g1[...])
        h1 = jnp.maximum(
            ar + g2[...] + _dot(ew[...].astype(F32), wew[...]) + be1[...], 0.0)
        e = _dot(h1, w2e[...]) + b2e[...]
        h2 = jnp.maximum(br + _dot(e, wn1e[...]) + bn1[...], 0.0)
        n1v = _dot(h2, wn2[...]) + bn2[...]
        if write_e:
            outs[0][...] = e.astype(jnp.bfloat16)
            outs[1][...] = n1v
        else:
            outs[0][...] = n1v

    out_specs = [pl.BlockSpec((BE, GD), lambda i: (i, 0))]
    out_shape = [jax.ShapeDtypeStruct((ee, GD), F32)]
    if write_e:
        out_specs = out_specs * 2
        out_shape = [jax.ShapeDtypeStruct((ee, GD), jnp.bfloat16)] + out_shape
    return pl.pallas_call(
        body,
        grid=(ee // BE,),
        in_specs=[pl.BlockSpec((BE, GD), lambda i: (i, 0)),
                  pl.BlockSpec((BE, GD), lambda i: (i, 0)),
                  pl.BlockSpec((BE, ewd), lambda i: (i, 0)),
                  _wspec(ewd), _wspec(GD), _wspec(GD), _wspec(GD),
                  _wspec(1), _wspec(1), _wspec(1), _wspec(1)],
        out_specs=out_specs,
        out_shape=out_shape,
    )


_tc_edges1 = tuple(_make_edge(pe, EID, True) for pe in PARTS)
_tc_edges2 = tuple(_make_edge(pe, GD, True) for pe in PARTS)
_tc_edges3 = tuple(_make_edge(pe, GD, False) for pe in PARTS)


def _make_node(first, last):
    def body(*refs):
        it = iter(refs)
        ps = [next(it) for _ in range(2 * len(PARTS))]
        if first:
            c0, c1 = next(it), next(it)
        else:
            inv_in = next(it)
        x_ref, wmx, wma, wm2, ubm, bm2 = (next(it) for _ in range(6))
        if not last:
            wr2, wb2, wc2 = next(it), next(it), next(it)
            xn_o, trow_o, tcol_o = next(it), next(it), next(it)
        mp_o = next(it)
        if first:
            inv_o = next(it)

        i = pl.program_id(0)
        vals = [p[0] for p in ps]
        while len(vals) > 1:
            vals = [a + b for a, b in zip(vals[::2], vals[1::2])]
        s = vals[0]
        if first:
            cnt = c0[0][:, :1] + c1[0][:, :1]
            invb = 1.0 / jnp.maximum(cnt, 1.0)
            inv_o[...] = jnp.broadcast_to(invb, (BN, GD))
            agg = s * invb
        else:
            agg = s * inv_in[...]
        x = x_ref[...]
        h = jnp.maximum(_dot(x, wmx[...]) + _dot(agg, wma[...]) + ubm[...], 0.0)
        xn = _dot(h, wm2[...]) + bm2[...]
        if not last:
            xn_o[...] = xn
            trow_o[...] = _pack2(_dot(xn, wr2[...]), _dot(xn, wb2[...]))
            tcol_o[...] = _dot(xn, wc2[...])

        @pl.when(i == 0)
        def _():
            mp_o[...] = jnp.zeros((8, GD), F32)

        mp_o[0:1, :] = mp_o[0:1, :] + jnp.sum(xn, axis=0, keepdims=True)

    p_spec = [pl.BlockSpec((1, BN, GD), lambda i: (0, i, 0)),
              pl.BlockSpec((1, BN, GD), lambda i: (1, i, 0))] * len(PARTS)
    in_specs = list(p_spec)
    if first:
        in_specs += [pl.BlockSpec((1, BN, GD), lambda i: (0, i, 0)),
                     pl.BlockSpec((1, BN, GD), lambda i: (1, i, 0))]
    else:
        in_specs += [pl.BlockSpec((BN, GD), lambda i: (i, 0))]
    in_specs += [pl.BlockSpec((BN, GD), lambda i: (i, 0)),
                 _wspec(GD), _wspec(GD), _wspec(GD), _wspec(1), _wspec(1)]
    if not last:
        in_specs += [_wspec(GD), _wspec(GD), _wspec(GD)]

    out_specs, out_shape = [], []
    if not last:
        out_specs += [pl.BlockSpec((BN, GD), lambda i: (i, 0)),
                      pl.BlockSpec((BN, GD), lambda i: (i, 0)),
                      pl.BlockSpec((BN, GD), lambda i: (i, 0))]
        out_shape += [jax.ShapeDtypeStruct((N, GD), F32),
                      jax.ShapeDtypeStruct((N, GD), F32),
                      jax.ShapeDtypeStruct((N, GD), F32)]
    out_specs += [pl.BlockSpec((8, GD), lambda i: (0, 0))]
    out_shape += [jax.ShapeDtypeStruct((8, GD), F32)]
    if first:
        out_specs += [pl.BlockSpec((BN, GD), lambda i: (i, 0))]
        out_shape += [jax.ShapeDtypeStruct((N, GD), F32)]
    return pl.pallas_call(body, grid=(GN,), in_specs=in_specs,
                          out_specs=out_specs, out_shape=out_shape)


_tc_node1 = _make_node(True, False)
_tc_node2 = _make_node(False, False)
_tc_node3 = _make_node(False, True)


def _make_glob(last):
    def body(mp, u, wgu, wgm, bg1, wg2, bg2, *rest):
        m = jnp.sum(mp[...], axis=0, keepdims=True) * (1.0 / N)
        m8 = jnp.broadcast_to(m, (8, GD))
        h = jnp.maximum(_dot(u[...], wgu[...]) + _dot(m8, wgm[...]) + bg1[...],
                        0.0)
        un = _dot(h, wg2[...]) + bg2[...]
        if last:
            wlin, blin, out_o = rest
            out_o[...] = (_dot(un, wlin[...]) + blin[...])[0:1, :]
        else:
            wue, b1e, wum, bm1, un_o, be1_o, ubm_o = rest
            un_o[...] = un
            be1_o[...] = (_dot(un, wue[...]) + b1e[...])[0:1, :]
            ubm_o[...] = (_dot(un, wum[...]) + bm1[...])[0:1, :]

    in_specs = [_wspec(8), _wspec(8), _wspec(GD), _wspec(GD), _wspec(1),
                _wspec(GD), _wspec(1)]
    if last:
        in_specs += [_wspec(GD), _wspec(1)]
        out_specs = [_wspec(1)]
        out_shape = [jax.ShapeDtypeStruct((1, GD), F32)]
    else:
        in_specs += [_wspec(GD), _wspec(1), _wspec(GD), _wspec(1)]
        out_specs = [_wspec(8), _wspec(1), _wspec(1)]
        out_shape = [jax.ShapeDtypeStruct((8, GD), F32),
                     jax.ShapeDtypeStruct((1, GD), F32),
                     jax.ShapeDtypeStruct((1, GD), F32)]
    return pl.pallas_call(body, grid=(1,), in_specs=in_specs,
                          out_specs=out_specs, out_shape=out_shape)


_tc_glob_mid = _make_glob(False)
_tc_glob_last = _make_glob(True)


def _split(p, ni, ewd, has_u):
    (w1e, b1e, w2e, b2e) = p['edge']
    (wn1, bn1, wn2, bn2) = p['nm1']
    (wm1, bm1, wm2, bm2) = p['nm2']
    (wg1, bg1, wg2, bg2) = p['glob']
    r = lambda b: b.reshape(1, GD)
    d = {
        'wr': w1e[:, :ni].T, 'wc': w1e[:, ni:2 * ni].T,
        'wew': w1e[:, 2 * ni:2 * ni + ewd].T,
        'b1e': r(b1e), 'w2e': w2e.T, 'b2e': r(b2e),
        'wbx': wn1[:, :ni].T, 'wn1e': wn1[:, ni:].T, 'bn1': r(bn1),
        'wn2': wn2.T, 'bn2': r(bn2),
        'wmx': wm1[:, :ni].T, 'wma': wm1[:, ni:ni + GD].T, 'bm1': r(bm1),
        'wm2': wm2.T, 'bm2': r(bm2),
        'wgu': wg1[:, :GD].T, 'wgm': wg1[:, GD:].T, 'bg1': r(bg1),
        'wg2': wg2.T, 'bg2': r(bg2),
    }
    if has_u:
        d['wue'] = w1e[:, 2 * ni + ewd:].T
        d['wum'] = wm1[:, ni + GD:].T
    return d


def kernel(x, edge_index, edge_w, params):
    _sc_gathers, _sc_count, _sc_scatters = _sc_kernels()
    row, col = edge_index[0], edge_index[1]
    q1 = _split(params['conv1'], GD, EID, False)
    q2 = _split(params['conv2'], GD, GD, True)
    q3 = _split(params['conv3'], GD, GD, True)
    wlin, blin = params['lin'][0].T, params['lin'][1].reshape(1, GD)

    u0 = jax.random.normal(jax.random.key(42), (1, GD), F32) * 0.01
    u0p = jnp.concatenate([u0, jnp.zeros((7, GD), F32)], axis=0)
    zz = jnp.zeros((NP, GD), F32)
    onesh = jnp.ones((CH, GD), F32)

    colr = col.reshape(NW, NCH, CH)
    rowrs, colrs, ews = [], [], []
    for pe, po in zip(PARTS, POFF):
        rowrs.append(lax.dynamic_slice(row, (po,), (pe,)).reshape(
            NW, pe // NW // CH, CH))
        colrs.append(lax.dynamic_slice(col, (po,), (pe,)).reshape(
            NW, pe // NW // CH, CH))
        ews.append(lax.dynamic_slice(edge_w, (po, 0), (pe, EID)))

    def layer(edges, trow, tcol, ew_parts, qq, be1):
        rs, pss = [], []
        for p in range(len(PARTS)):
            g1, g2 = _sc_gathers[p](trow, tcol, rowrs[p], colrs[p])
            rs.append(edges[p](g1, g2, ew_parts[p], qq['wew'], qq['w2e'],
                               qq['wn1e'], qq['wn2'], be1, qq['b2e'],
                               qq['bn1'], qq['bn2']))
        for p in range(len(PARTS)):
            pss.append(_sc_scatters[p](rs[p][-1], colrs[p], zz))
        e_parts = tuple(r[0] for r in rs) if len(rs[0]) == 2 else None
        return e_parts, pss

    def pargs(pss):
        out = []
        for ps in pss:
            r = ps.reshape(2, NP, GD)
            out += [r, r]
        return out

    trow, tcol = _tc_proj(x, q1['wr'], q1['wbx'], q1['wc'])
    pc = _sc_count(colr, zz, onesh)
    e1, pss = layer(_tc_edges1, trow, tcol, ews, q1, q1['b1e'])
    xn1, trow, tcol, mp, inv = _tc_node1(
        *pargs(pss), pc.reshape(2, NP, GD), pc.reshape(2, NP, GD),
        x, q1['wmx'], q1['wma'], q1['wm2'], q1['bm1'], q1['bm2'],
        q2['wr'], q2['wbx'], q2['wc'])
    u1, be1_2, ubm2 = _tc_glob_mid(mp, u0p, q1['wgu'], q1['wgm'], q1['bg1'],
                                   q1['wg2'], q1['bg2'],
                                   q2['wue'], q2['b1e'], q2['wum'], q2['bm1'])

    e2, pss = layer(_tc_edges2, trow, tcol, e1, q2, be1_2)
    xn2, trow, tcol, mp = _tc_node2(
        *pargs(pss), inv,
        xn1, q2['wmx'], q2['wma'], q2['wm2'], ubm2, q2['bm2'],
        q3['wr'], q3['wbx'], q3['wc'])
    u2, be1_3, ubm3 = _tc_glob_mid(mp, u1, q2['wgu'], q2['wgm'], q2['bg1'],
                                   q2['wg2'], q2['bg2'],
                                   q3['wue'], q3['b1e'], q3['wum'], q3['bm1'])

    _, pss = layer(_tc_edges3, trow, tcol, e2, q3, be1_3)
    (mp,) = _tc_node3(
        *pargs(pss), inv,
        xn2, q3['wmx'], q3['wma'], q3['wm2'], ubm3, q3['bm2'])
    (out,) = _tc_glob_last(mp, u2, q3['wgu'], q3['wgm'], q3['bg1'],
                           q3['wg2'], q3['bg2'], wlin, blin)
    return out
